# baseline (speedup 1.0000x reference)
import functools
import jax
import jax.numpy as jnp
from jax import lax
from jax.experimental import pallas as pl
from jax.experimental.pallas import tpu as pltpu

N_NODES = 50000
N_SUB = 150000
E_SUB = 480000
D = 128
P = 64

NP = 51200
BN = 1024
GRID = NP // BN


def _tc_reduce_body(g_ref, bmat_ref, ct_ref, x_ref, s2x_ref,
                    w0_ref, b0_ref, wu_ref, bu_ref,
                    mc_ref, sg_ref, p1_ref, mcct_ref, smalls_ref, r_ref):
    i = pl.program_id(0)

    gblk = g_ref[...]
    bblk = bmat_ref[...]
    ctblk = ct_ref[...]
    xblk = x_ref[...]
    s2xblk = s2x_ref[...]

    mblk = gblk + bblk
    cm = jnp.maximum(jnp.sum(ctblk, axis=0, keepdims=True), 1.0)
    mcblk = mblk / cm
    mc_ref[...] = mcblk

    sg = jnp.dot(mblk, xblk, preferred_element_type=jnp.float32)
    p1 = jnp.dot(mcblk, s2xblk, preferred_element_type=jnp.float32)
    mcct = lax.dot_general(mcblk, ctblk, (((1,), (1,)), ((), ())),
                           preferred_element_type=jnp.float32)

    lane = lax.broadcasted_iota(jnp.int32, (P, D), 1)
    rs_m = jnp.sum(mblk, axis=1)
    rs_ct = jnp.sum(ctblk, axis=1)
    smalls = (jnp.where(lane == 0, rs_m[:, None], 0.0)
              + jnp.where(lane == 1, rs_ct[:, None], 0.0))

    @pl.when(i == 0)
    def _init():
        sg_ref[...] = sg
        p1_ref[...] = p1
        mcct_ref[...] = mcct
        smalls_ref[...] = smalls

    @pl.when(i > 0)
    def _acc():
        sg_ref[...] += sg
        p1_ref[...] += p1
        mcct_ref[...] += mcct
        smalls_ref[...] += smalls

    @pl.when(i == GRID - 1)
    def _epilogue():
        sm = smalls_ref[...]
        cnt_b = jnp.sum(jnp.where(lane == 1, sm, 0.0), axis=1)
        cb = jnp.maximum(cnt_b, 1.0)
        sgpool = (jnp.dot(sg_ref[...] / cb[:, None], w0_ref[...],
                          preferred_element_type=jnp.float32)
                  + b0_ref[...])
        r_ref[...] = jax.nn.relu(
            jnp.dot(sgpool, wu_ref[...], preferred_element_type=jnp.float32)
            + bu_ref[...])


def _tc_reduce(G, B, Ct, x, s2x, W0, b0, Wu, bu):
    grid = (GRID,)
    tbl = pl.BlockSpec((P, BN), lambda i: (0, i))
    xsp = pl.BlockSpec((BN, D), lambda i: (i, 0))
    wsp = pl.BlockSpec((D, D), lambda i: (0, 0))
    bsp = pl.BlockSpec((1, D), lambda i: (0, 0))
    acc = pl.BlockSpec((P, D), lambda i: (0, 0))
    out_shapes = (
        jax.ShapeDtypeStruct((P, NP), jnp.float32),
        jax.ShapeDtypeStruct((P, D), jnp.float32),
        jax.ShapeDtypeStruct((P, D), jnp.float32),
        jax.ShapeDtypeStruct((P, P), jnp.float32),
        jax.ShapeDtypeStruct((P, D), jnp.float32),
        jax.ShapeDtypeStruct((P, D), jnp.float32),
    )
    return pl.pallas_call(
        _tc_reduce_body,
        grid=grid,
        in_specs=[tbl, tbl, tbl, xsp, xsp, wsp, bsp, wsp, bsp],
        out_specs=(tbl, acc, acc, pl.BlockSpec((P, P), lambda i: (0, 0)),
                   acc, acc),
        out_shape=out_shapes,
        compiler_params=pltpu.CompilerParams(
            dimension_semantics=("arbitrary",)),
    )(G, B, Ct, x, s2x, W0, b0, Wu, bu)


def _tc_out_body(k_ref, x_ref, p1_ref, mcct_ref, r_ref, smalls_ref,
                 w0_ref, b0_ref, w1_ref, b1_ref,
                 kx_ref, out_ref):
    i = pl.program_id(0)
    kx = jnp.dot(k_ref[...], x_ref[...], preferred_element_type=jnp.float32)

    @pl.when(i == 0)
    def _init():
        kx_ref[...] = kx

    @pl.when(i > 0)
    def _acc():
        kx_ref[...] += kx

    @pl.when(i == GRID - 1)
    def _epilogue():
        lane = lax.broadcasted_iota(jnp.int32, (P, D), 1)
        sm = smalls_ref[...]
        colsum_m = jnp.sum(jnp.where(lane == 0, sm, 0.0), axis=1)
        cnt_b = jnp.sum(jnp.where(lane == 1, sm, 0.0), axis=1)
        cb = jnp.maximum(cnt_b, 1.0)
        out_sum = (jnp.dot(kx_ref[...] + p1_ref[...], w0_ref[...],
                           preferred_element_type=jnp.float32)
                   + jnp.dot(mcct_ref[...], r_ref[...],
                             preferred_element_type=jnp.float32)
                   + colsum_m[:, None] * b0_ref[...])
        out = (jnp.dot(out_sum / cb[:, None], w1_ref[...],
                       preferred_element_type=jnp.float32)
               + b1_ref[...])
        out_ref[...] = jnp.where(cnt_b[:, None] > 0, out, 0.0)


def _tc_out(K, x, partial1, McCt, r, smalls, W0, b0, W1, b1):
    tbl = pl.BlockSpec((P, BN), lambda i: (0, i))
    xsp = pl.BlockSpec((BN, D), lambda i: (i, 0))
    wsp = pl.BlockSpec((D, D), lambda i: (0, 0))
    bsp = pl.BlockSpec((1, D), lambda i: (0, 0))
    acc = pl.BlockSpec((P, D), lambda i: (0, 0))
    out_shapes = (
        jax.ShapeDtypeStruct((P, D), jnp.float32),
        jax.ShapeDtypeStruct((P, D), jnp.float32),
    )
    res = pl.pallas_call(
        _tc_out_body,
        grid=(GRID,),
        in_specs=[tbl, xsp, acc, pl.BlockSpec((P, P), lambda i: (0, 0)),
                  acc, acc, wsp, bsp, wsp, bsp],
        out_specs=(acc, acc),
        out_shape=out_shapes,
        compiler_params=pltpu.CompilerParams(
            dimension_semantics=("arbitrary",)),
    )(K, x, partial1, McCt, r, smalls, W0, b0, W1, b1)
    return res[1]


def kernel(x, edge_attr, subgraphs_nodes_mapper, subgraphs_edges_mapper,
           combined_subgraphs, subgraphs_batch, W0, b0, W1, b1, Wu, bu):
    m = subgraphs_nodes_mapper
    b = subgraphs_batch
    src = combined_subgraphs[0]
    dst = combined_subgraphs[1]

    indeg = jax.ops.segment_sum(jnp.ones((E_SUB,), jnp.float32), dst,
                                num_segments=N_SUB)
    deg = 1.0 + indeg
    dinv = lax.rsqrt(deg)
    dinv2 = 1.0 / deg

    ms = m[src]
    md = m[dst]
    bd = b[dst]
    w = dinv[src] * dinv[dst]

    G = jax.ops.segment_sum(w, bd * NP + ms, num_segments=P * NP).reshape(P, NP)
    B = jax.ops.segment_sum(dinv2, b * NP + m, num_segments=P * NP).reshape(P, NP)
    Ct = jax.ops.segment_sum(jnp.ones((N_SUB,), jnp.float32), b * NP + m,
                             num_segments=P * NP).reshape(P, NP)
    s2 = jax.ops.segment_sum(dinv2, m, num_segments=NP)

    xp = jnp.pad(x, ((0, NP - N_NODES), (0, 0)))
    s2x = xp * s2[:, None]

    Mc, sg_sum, partial1, McCt, smalls, r = _tc_reduce(
        G, B, Ct, xp, s2x, W0, b0[None, :], Wu, bu[None, :])

    K = jax.ops.segment_sum(
        (w[:, None] * Mc[:, md].T).reshape(E_SUB * P),
        (ms[:, None] * P + jnp.arange(P, dtype=jnp.int32)[None, :]).reshape(E_SUB * P),
        num_segments=NP * P).reshape(NP, P).T

    return _tc_out(K, xp, partial1, McCt, r, smalls, W0, b0[None, :],
                   W1, b1[None, :])


# speedup vs baseline: 12.5278x; 12.5278x over previous
import functools
import jax
import jax.numpy as jnp
from jax import lax
from jax.experimental import pallas as pl
from jax.experimental.pallas import tpu as pltpu
from jax.experimental.pallas import tpu_sc as plsc

N_NODES = 50000
N_SUB = 150000
E_SUB = 480000
D = 128
P = 64

NP = 51200
BN = 1024
GRID = NP // BN

E_PAD = 491520
EROWS = E_PAD // 128
EW_ROWS = EROWS // 32
S_TBL = 150016
S_STREAM = 150528
SROWS = S_STREAM // 128
KW_I = S_TBL // 32
KW_S = NP // 32

_MESH = plsc.VectorSubcoreMesh(core_axis_name="c", subcore_axis_name="s",
                               num_cores=2, num_subcores=16)


def _wid():
    return lax.axis_index("c") * 16 + lax.axis_index("s")


def _sc_gather_maps_body(src_hbm, dst_hbm, mtbl_hbm, btbl_hbm, zeros_hbm,
                         ms_hbm, md_hbm, bd_hbm, indeg_hbm,
                         m_sh, b_sh, sbuf, dbuf, msv, mdv, bdv, acc, sem):
    s = lax.axis_index("s")
    wid = _wid()

    @pl.when(s == 0)
    def _load():
        pltpu.sync_copy(mtbl_hbm, m_sh)
        pltpu.sync_copy(btbl_hbm, b_sh)

    plsc.subcore_barrier()

    rowbase = wid * EW_ROWS

    def chunk1(ci, carry):
        ro = rowbase + ci * 8
        pltpu.sync_copy(src_hbm.at[pl.ds(ro, 8)], sbuf)
        pltpu.sync_copy(dst_hbm.at[pl.ds(ro, 8)], dbuf)
        hs = []
        for j in range(8):
            hs.append(pltpu.async_copy(m_sh.at[sbuf.at[j]], msv.at[j], sem))
            hs.append(pltpu.async_copy(m_sh.at[dbuf.at[j]], mdv.at[j], sem))
            hs.append(pltpu.async_copy(b_sh.at[dbuf.at[j]], bdv.at[j], sem))
        for h in hs:
            h.wait()
        pltpu.sync_copy(msv, ms_hbm.at[pl.ds(ro, 8)])
        pltpu.sync_copy(mdv, md_hbm.at[pl.ds(ro, 8)])
        pltpu.sync_copy(bdv, bd_hbm.at[pl.ds(ro, 8)])
        return carry

    lax.fori_loop(0, EW_ROWS // 8, chunk1, 0)

    kbase = wid * KW_I
    pltpu.sync_copy(zeros_hbm.at[pl.ds(0, KW_I)], acc)
    ones16 = jnp.ones((16,), jnp.float32)

    def chunk2(ci, carry):
        pltpu.sync_copy(dst_hbm.at[pl.ds(ci * 8, 8)], dbuf)
        for j in range(8):
            def inner(l, c2):
                d16 = dbuf[j, pl.ds(l * 16, 16)]
                loc = d16 - kbase
                msk = (d16 >= kbase) & (d16 < kbase + KW_I)
                locc = jnp.minimum(jnp.maximum(loc, 0), KW_I - 1)
                plsc.addupdate_scatter(acc, [locc], ones16, mask=msk)
                return c2
            lax.fori_loop(0, 8, inner, 0)
        return carry

    lax.fori_loop(0, EROWS // 8, chunk2, 0)
    pltpu.sync_copy(acc, indeg_hbm.at[pl.ds(kbase, KW_I)])


def _sc_gather_maps(src2, dst2, m_tbl, b_tbl, zeros_np):
    f = pl.kernel(
        _sc_gather_maps_body,
        out_type=(
            jax.ShapeDtypeStruct((EROWS, 128), jnp.int32),
            jax.ShapeDtypeStruct((EROWS, 128), jnp.int32),
            jax.ShapeDtypeStruct((EROWS, 128), jnp.int32),
            jax.ShapeDtypeStruct((S_TBL,), jnp.float32),
        ),
        mesh=_MESH,
        compiler_params=pltpu.CompilerParams(needs_layout_passes=False),
        scratch_types=[
            pltpu.VMEM_SHARED((S_TBL,), jnp.int32),
            pltpu.VMEM_SHARED((S_TBL,), jnp.int32),
            pltpu.VMEM((8, 128), jnp.int32),
            pltpu.VMEM((8, 128), jnp.int32),
            pltpu.VMEM((8, 128), jnp.int32),
            pltpu.VMEM((8, 128), jnp.int32),
            pltpu.VMEM((8, 128), jnp.int32),
            pltpu.VMEM((KW_I,), jnp.float32),
            pltpu.SemaphoreType.DMA,
        ],
    )
    return f(src2, dst2, m_tbl, b_tbl, zeros_np)


def _sc_edge_weights_body(src_hbm, dst_hbm, dinv_hbm, ms2_hbm, dv2s_hbm,
                          zeros_hbm,
                          w_hbm, s2_hbm,
                          dinv_sh, sbuf, dbuf, va, vb, wbuf, mbuf, vbuf,
                          acc, sem):
    s = lax.axis_index("s")
    wid = _wid()

    @pl.when(s == 0)
    def _load():
        pltpu.sync_copy(dinv_hbm, dinv_sh)

    plsc.subcore_barrier()

    rowbase = wid * EW_ROWS

    def chunk1(ci, carry):
        ro = rowbase + ci * 8
        pltpu.sync_copy(src_hbm.at[pl.ds(ro, 8)], sbuf)
        pltpu.sync_copy(dst_hbm.at[pl.ds(ro, 8)], dbuf)
        hs = []
        for j in range(8):
            hs.append(pltpu.async_copy(dinv_sh.at[sbuf.at[j]], va.at[j], sem))
            hs.append(pltpu.async_copy(dinv_sh.at[dbuf.at[j]], vb.at[j], sem))
        for h in hs:
            h.wait()
        for j in range(8):
            def inner(l, c2):
                sl = pl.ds(l * 16, 16)
                wbuf[j, sl] = va[j, sl] * vb[j, sl]
                return c2
            lax.fori_loop(0, 8, inner, 0)
        pltpu.sync_copy(wbuf, w_hbm.at[pl.ds(ro, 8)])
        return carry

    lax.fori_loop(0, EW_ROWS // 8, chunk1, 0)

    kbase = wid * KW_S
    pltpu.sync_copy(zeros_hbm.at[pl.ds(0, KW_S)], acc)

    def chunk2(ci, carry):
        pltpu.sync_copy(ms2_hbm.at[pl.ds(ci * 8, 8)], mbuf)
        pltpu.sync_copy(dv2s_hbm.at[pl.ds(ci * 8, 8)], vbuf)
        for j in range(8):
            def inner(l, c2):
                sl = pl.ds(l * 16, 16)
                i16 = mbuf[j, sl]
                v16 = vbuf[j, sl]
                loc = i16 - kbase
                msk = (i16 >= kbase) & (i16 < kbase + KW_S)
                locc = jnp.minimum(jnp.maximum(loc, 0), KW_S - 1)
                plsc.addupdate_scatter(acc, [locc], v16, mask=msk)
                return c2
            lax.fori_loop(0, 8, inner, 0)
        return carry

    lax.fori_loop(0, SROWS // 8, chunk2, 0)
    pltpu.sync_copy(acc, s2_hbm.at[pl.ds(kbase, KW_S)])


def _sc_edge_weights(src2, dst2, dinv_tbl, m_s2, dv2_s2, zeros_np):
    f = pl.kernel(
        _sc_edge_weights_body,
        out_type=(
            jax.ShapeDtypeStruct((EROWS, 128), jnp.float32),
            jax.ShapeDtypeStruct((NP,), jnp.float32),
        ),
        mesh=_MESH,
        compiler_params=pltpu.CompilerParams(needs_layout_passes=False),
        scratch_types=[
            pltpu.VMEM_SHARED((S_TBL,), jnp.float32),
            pltpu.VMEM((8, 128), jnp.int32),
            pltpu.VMEM((8, 128), jnp.int32),
            pltpu.VMEM((8, 128), jnp.float32),
            pltpu.VMEM((8, 128), jnp.float32),
            pltpu.VMEM((8, 128), jnp.float32),
            pltpu.VMEM((8, 128), jnp.int32),
            pltpu.VMEM((8, 128), jnp.float32),
            pltpu.VMEM((KW_S,), jnp.float32),
            pltpu.SemaphoreType.DMA,
        ],
    )
    return f(src2, dst2, dinv_tbl, m_s2, dv2_s2, zeros_np)


def _sc_table_body(nrows, ki_hbm, kp_hbm, val_hbm, zeros_hbm, t_hbm,
                   acc0, acc1, ib, pb, vb):
    wid = _wid()
    p0 = 2 * wid
    pltpu.sync_copy(zeros_hbm, acc0)
    pltpu.sync_copy(zeros_hbm, acc1)

    def chunk(ci, carry):
        ro = ci * 8
        pltpu.sync_copy(ki_hbm.at[pl.ds(ro, 8)], ib)
        pltpu.sync_copy(kp_hbm.at[pl.ds(ro, 8)], pb)
        pltpu.sync_copy(val_hbm.at[pl.ds(ro, 8)], vb)
        for j in range(8):
            def inner(l, c2):
                sl = pl.ds(l * 16, 16)
                i16 = ib[j, sl]
                p16 = pb[j, sl]
                v16 = vb[j, sl]
                plsc.addupdate_scatter(acc0, [i16], v16, mask=p16 == p0)
                plsc.addupdate_scatter(acc1, [i16], v16, mask=p16 == p0 + 1)
                return c2
            lax.fori_loop(0, 8, inner, 0)
        return carry

    lax.fori_loop(0, nrows // 8, chunk, 0)
    pltpu.sync_copy(acc0, t_hbm.at[pl.ds(p0 * NP, NP)])
    pltpu.sync_copy(acc1, t_hbm.at[pl.ds((p0 + 1) * NP, NP)])


def _sc_table(ki2, kp2, val2, nrows, zeros_np):
    f = pl.kernel(
        functools.partial(_sc_table_body, nrows),
        out_type=jax.ShapeDtypeStruct((P * NP,), jnp.float32),
        mesh=_MESH,
        compiler_params=pltpu.CompilerParams(needs_layout_passes=False),
        scratch_types=[
            pltpu.VMEM((NP,), jnp.float32),
            pltpu.VMEM((NP,), jnp.float32),
            pltpu.VMEM((8, 128), jnp.int32),
            pltpu.VMEM((8, 128), jnp.int32),
            pltpu.VMEM((8, 128), jnp.float32),
        ],
    )
    return f(ki2, kp2, val2, zeros_np).reshape(P, NP)


def _sc_kbuild_body(md_hbm, ms_hbm, w_hbm, mc_hbm, zeros_hbm, k_hbm,
                    mc_sh, mdb, msb, wb, idx0, g0, acc0, sem):
    c = lax.axis_index("c")
    s = lax.axis_index("s")

    for h in range(2):
        @pl.when(s == 0)
        def _load():
            pltpu.sync_copy(
                mc_hbm.at[pl.ds((c * 32 + h * 16) * NP, 16 * NP)], mc_sh)

        plsc.subcore_barrier()

        pltpu.sync_copy(zeros_hbm, acc0)
        off0 = s * NP
        p0 = c * 32 + h * 16 + s

        def chunk(ci, carry):
            ro = ci * 8
            pltpu.sync_copy(md_hbm.at[pl.ds(ro, 8)], mdb)
            pltpu.sync_copy(ms_hbm.at[pl.ds(ro, 8)], msb)
            pltpu.sync_copy(w_hbm.at[pl.ds(ro, 8)], wb)
            for j in range(8):
                def inner(l, c2):
                    sl = pl.ds(l * 16, 16)
                    idx0[j, sl] = mdb[j, sl] + off0
                    return c2
                lax.fori_loop(0, 8, inner, 0)
            hs = []
            for j in range(8):
                hs.append(
                    pltpu.async_copy(mc_sh.at[idx0.at[j]], g0.at[j], sem))
            for hh in hs:
                hh.wait()
            for j in range(8):
                def inner2(l, c2):
                    sl = pl.ds(l * 16, 16)
                    i16 = msb[j, sl]
                    v16 = wb[j, sl]
                    plsc.addupdate_scatter(acc0, [i16], v16 * g0[j, sl])
                    return c2
                lax.fori_loop(0, 8, inner2, 0)
            return carry

        lax.fori_loop(0, EROWS // 8, chunk, 0)
        pltpu.sync_copy(acc0, k_hbm.at[pl.ds(p0 * NP, NP)])
        plsc.subcore_barrier()


def _sc_kbuild(md2, ms2, w2, mc_flat, zeros_np):
    f = pl.kernel(
        _sc_kbuild_body,
        out_type=jax.ShapeDtypeStruct((P * NP,), jnp.float32),
        mesh=_MESH,
        compiler_params=pltpu.CompilerParams(needs_layout_passes=False),
        scratch_types=[
            pltpu.VMEM_SHARED((16 * NP,), jnp.float32),
            pltpu.VMEM((8, 128), jnp.int32),
            pltpu.VMEM((8, 128), jnp.int32),
            pltpu.VMEM((8, 128), jnp.float32),
            pltpu.VMEM((8, 128), jnp.int32),
            pltpu.VMEM((8, 128), jnp.float32),
            pltpu.VMEM((NP,), jnp.float32),
            pltpu.SemaphoreType.DMA,
        ],
    )
    return f(md2, ms2, w2, mc_flat, zeros_np).reshape(P, NP)


def _tc_reduce_body(g_ref, bmat_ref, ct_ref, x_ref, s2x_ref,
                    w0_ref, b0_ref, wu_ref, bu_ref,
                    mc_ref, sg_ref, p1_ref, mcct_ref, smalls_ref, r_ref):
    i = pl.program_id(0)

    gblk = g_ref[...]
    bblk = bmat_ref[...]
    ctblk = ct_ref[...]
    xblk = x_ref[...]
    s2xblk = s2x_ref[...]

    mblk = gblk + bblk
    cm = jnp.maximum(jnp.sum(ctblk, axis=0, keepdims=True), 1.0)
    mcblk = mblk / cm
    mc_ref[...] = mcblk

    sg = jnp.dot(mblk, xblk, preferred_element_type=jnp.float32)
    p1 = jnp.dot(mcblk, s2xblk, preferred_element_type=jnp.float32)
    mcct = lax.dot_general(mcblk, ctblk, (((1,), (1,)), ((), ())),
                           preferred_element_type=jnp.float32)

    lane = lax.broadcasted_iota(jnp.int32, (P, D), 1)
    rs_m = jnp.sum(mblk, axis=1)
    rs_ct = jnp.sum(ctblk, axis=1)
    smalls = (jnp.where(lane == 0, rs_m[:, None], 0.0)
              + jnp.where(lane == 1, rs_ct[:, None], 0.0))

    @pl.when(i == 0)
    def _init():
        sg_ref[...] = sg
        p1_ref[...] = p1
        mcct_ref[...] = mcct
        smalls_ref[...] = smalls

    @pl.when(i > 0)
    def _acc():
        sg_ref[...] += sg
        p1_ref[...] += p1
        mcct_ref[...] += mcct
        smalls_ref[...] += smalls

    @pl.when(i == GRID - 1)
    def _epilogue():
        sm = smalls_ref[...]
        cnt_b = jnp.sum(jnp.where(lane == 1, sm, 0.0), axis=1)
        cb = jnp.maximum(cnt_b, 1.0)
        sgpool = (jnp.dot(sg_ref[...] / cb[:, None], w0_ref[...],
                          preferred_element_type=jnp.float32)
                  + b0_ref[...])
        r_ref[...] = jax.nn.relu(
            jnp.dot(sgpool, wu_ref[...], preferred_element_type=jnp.float32)
            + bu_ref[...])


def _tc_reduce(G, B, Ct, x, s2x, W0, b0, Wu, bu):
    tbl = pl.BlockSpec((P, BN), lambda i: (0, i))
    xsp = pl.BlockSpec((BN, D), lambda i: (i, 0))
    wsp = pl.BlockSpec((D, D), lambda i: (0, 0))
    bsp = pl.BlockSpec((1, D), lambda i: (0, 0))
    acc = pl.BlockSpec((P, D), lambda i: (0, 0))
    out_shapes = (
        jax.ShapeDtypeStruct((P, NP), jnp.float32),
        jax.ShapeDtypeStruct((P, D), jnp.float32),
        jax.ShapeDtypeStruct((P, D), jnp.float32),
        jax.ShapeDtypeStruct((P, P), jnp.float32),
        jax.ShapeDtypeStruct((P, D), jnp.float32),
        jax.ShapeDtypeStruct((P, D), jnp.float32),
    )
    return pl.pallas_call(
        _tc_reduce_body,
        grid=(GRID,),
        in_specs=[tbl, tbl, tbl, xsp, xsp, wsp, bsp, wsp, bsp],
        out_specs=(tbl, acc, acc, pl.BlockSpec((P, P), lambda i: (0, 0)),
                   acc, acc),
        out_shape=out_shapes,
        compiler_params=pltpu.CompilerParams(
            dimension_semantics=("arbitrary",)),
    )(G, B, Ct, x, s2x, W0, b0, Wu, bu)


def _tc_out_body(k_ref, x_ref, p1_ref, mcct_ref, r_ref, smalls_ref,
                 w0_ref, b0_ref, w1_ref, b1_ref,
                 kx_ref, out_ref):
    i = pl.program_id(0)
    kx = jnp.dot(k_ref[...], x_ref[...], preferred_element_type=jnp.float32)

    @pl.when(i == 0)
    def _init():
        kx_ref[...] = kx

    @pl.when(i > 0)
    def _acc():
        kx_ref[...] += kx

    @pl.when(i == GRID - 1)
    def _epilogue():
        lane = lax.broadcasted_iota(jnp.int32, (P, D), 1)
        sm = smalls_ref[...]
        colsum_m = jnp.sum(jnp.where(lane == 0, sm, 0.0), axis=1)
        cnt_b = jnp.sum(jnp.where(lane == 1, sm, 0.0), axis=1)
        cb = jnp.maximum(cnt_b, 1.0)
        out_sum = (jnp.dot(kx_ref[...] + p1_ref[...], w0_ref[...],
                           preferred_element_type=jnp.float32)
                   + jnp.dot(mcct_ref[...], r_ref[...],
                             preferred_element_type=jnp.float32)
                   + colsum_m[:, None] * b0_ref[...])
        out = (jnp.dot(out_sum / cb[:, None], w1_ref[...],
                       preferred_element_type=jnp.float32)
               + b1_ref[...])
        out_ref[...] = jnp.where(cnt_b[:, None] > 0, out, 0.0)


def _tc_out(K, x, partial1, McCt, r, smalls, W0, b0, W1, b1):
    tbl = pl.BlockSpec((P, BN), lambda i: (0, i))
    xsp = pl.BlockSpec((BN, D), lambda i: (i, 0))
    wsp = pl.BlockSpec((D, D), lambda i: (0, 0))
    bsp = pl.BlockSpec((1, D), lambda i: (0, 0))
    acc = pl.BlockSpec((P, D), lambda i: (0, 0))
    out_shapes = (
        jax.ShapeDtypeStruct((P, D), jnp.float32),
        jax.ShapeDtypeStruct((P, D), jnp.float32),
    )
    res = pl.pallas_call(
        _tc_out_body,
        grid=(GRID,),
        in_specs=[tbl, xsp, acc, pl.BlockSpec((P, P), lambda i: (0, 0)),
                  acc, acc, wsp, bsp, wsp, bsp],
        out_specs=(acc, acc),
        out_shape=out_shapes,
        compiler_params=pltpu.CompilerParams(
            dimension_semantics=("arbitrary",)),
    )(K, x, partial1, McCt, r, smalls, W0, b0, W1, b1)
    return res[1]


def kernel(x, edge_attr, subgraphs_nodes_mapper, subgraphs_edges_mapper,
           combined_subgraphs, subgraphs_batch, W0, b0, W1, b1, Wu, bu):
    m = subgraphs_nodes_mapper
    b = subgraphs_batch
    src = combined_subgraphs[0]
    dst = combined_subgraphs[1]

    zeros_np = jnp.zeros((NP,), jnp.float32)

    src2 = jnp.pad(src, (0, E_PAD - E_SUB),
                   constant_values=N_SUB).reshape(EROWS, 128)
    dst2 = jnp.pad(dst, (0, E_PAD - E_SUB),
                   constant_values=N_SUB).reshape(EROWS, 128)
    m_tbl = jnp.pad(m, (0, S_TBL - N_SUB), constant_values=NP - 1)
    b_tbl = jnp.pad(b, (0, S_TBL - N_SUB), constant_values=-1)

    ms2, md2, bd2, indeg = _sc_gather_maps(src2, dst2, m_tbl, b_tbl, zeros_np)

    deg = 1.0 + indeg[:N_SUB]
    dinv = lax.rsqrt(deg)
    dinv2 = 1.0 / deg
    dinv_tbl = jnp.pad(dinv, (0, S_TBL - N_SUB))

    m_s2 = jnp.pad(m, (0, S_STREAM - N_SUB),
                   constant_values=-1).reshape(SROWS, 128)
    b_s2 = jnp.pad(b, (0, S_STREAM - N_SUB),
                   constant_values=-1).reshape(SROWS, 128)
    dv2_s2 = jnp.pad(dinv2, (0, S_STREAM - N_SUB)).reshape(SROWS, 128)
    ones_s2 = jnp.ones((SROWS, 128), jnp.float32)

    w2, s2 = _sc_edge_weights(src2, dst2, dinv_tbl, m_s2, dv2_s2, zeros_np)

    G = _sc_table(ms2, bd2, w2, EROWS, zeros_np)
    Bm = _sc_table(m_s2, b_s2, dv2_s2, SROWS, zeros_np)
    Ct = _sc_table(m_s2, b_s2, ones_s2, SROWS, zeros_np)

    xp = jnp.pad(x, ((0, NP - N_NODES), (0, 0)))
    s2x = xp * s2[:, None]

    Mc, sg_sum, partial1, McCt, smalls, r = _tc_reduce(
        G, Bm, Ct, xp, s2x, W0, b0[None, :], Wu, bu[None, :])

    K = _sc_kbuild(md2, ms2, w2, Mc.reshape(P * NP), zeros_np)

    return _tc_out(K, xp, partial1, McCt, r, smalls, W0, b0[None, :],
                   W1, b1[None, :])


# speedup vs baseline: 19.5808x; 1.5630x over previous
import functools
import jax
import jax.numpy as jnp
from jax import lax
from jax.experimental import pallas as pl
from jax.experimental.pallas import tpu as pltpu
from jax.experimental.pallas import tpu_sc as plsc

N_NODES = 50000
N_SUB = 150000
E_SUB = 480000
D = 128
P = 64

NP = 51200
BN = 1024
GRID = NP // BN

E_PAD = 491520
EROWS = E_PAD // 128
EW_ROWS = EROWS // 32
S_TBL = 150016
S_STREAM = 150528
SROWS = S_STREAM // 128
KW_I = S_TBL // 32
KW_S = NP // 32
_CR = 24

_MESH = plsc.VectorSubcoreMesh(core_axis_name="c", subcore_axis_name="s",
                               num_cores=2, num_subcores=16)


def _wid():
    return lax.axis_index("c") * 16 + lax.axis_index("s")


def _sc_gather_maps_body(src_hbm, dst_hbm, mtbl_hbm, btbl_hbm, zeros_hbm,
                         ms_hbm, md_hbm, bd_hbm, indeg_hbm,
                         m_sh, b_sh, sbuf, dbuf, msv, mdv, bdv, acc, sem):
    s = lax.axis_index("s")
    wid = _wid()

    @pl.when(s == 0)
    def _load():
        pltpu.sync_copy(mtbl_hbm, m_sh)
        pltpu.sync_copy(btbl_hbm, b_sh)

    plsc.subcore_barrier()

    rowbase = wid * EW_ROWS

    def chunk1(ci, carry):
        ro = rowbase + ci * _CR
        pltpu.sync_copy(src_hbm.at[pl.ds(ro, _CR)], sbuf)
        pltpu.sync_copy(dst_hbm.at[pl.ds(ro, _CR)], dbuf)
        hs = []
        for j in range(_CR):
            hs.append(pltpu.async_copy(m_sh.at[sbuf.at[j]], msv.at[j], sem))
            hs.append(pltpu.async_copy(m_sh.at[dbuf.at[j]], mdv.at[j], sem))
            hs.append(pltpu.async_copy(b_sh.at[dbuf.at[j]], bdv.at[j], sem))
        for h in hs:
            h.wait()
        pltpu.sync_copy(msv, ms_hbm.at[pl.ds(ro, _CR)])
        pltpu.sync_copy(mdv, md_hbm.at[pl.ds(ro, _CR)])
        pltpu.sync_copy(bdv, bd_hbm.at[pl.ds(ro, _CR)])
        return carry

    lax.fori_loop(0, EW_ROWS // _CR, chunk1, 0)

    kbase = wid * KW_I
    pltpu.sync_copy(zeros_hbm.at[pl.ds(0, KW_I)], acc)
    ones16 = jnp.ones((16,), jnp.float32)

    def chunk2(ci, carry):
        pltpu.sync_copy(dst_hbm.at[pl.ds(ci * _CR, _CR)], dbuf)
        for j in range(_CR):
            for l in range(8):
                d16 = dbuf[j, pl.ds(l * 16, 16)]
                loc = d16 - kbase
                msk = (d16 >= kbase) & (d16 < kbase + KW_I)
                locc = jnp.minimum(jnp.maximum(loc, 0), KW_I - 1)
                plsc.addupdate_scatter(acc, [locc], ones16, mask=msk)
        return carry

    lax.fori_loop(0, EROWS // _CR, chunk2, 0)
    pltpu.sync_copy(acc, indeg_hbm.at[pl.ds(kbase, KW_I)])


def _sc_gather_maps(src2, dst2, m_tbl, b_tbl, zeros_np):
    f = pl.kernel(
        _sc_gather_maps_body,
        out_type=(
            jax.ShapeDtypeStruct((EROWS, 128), jnp.int32),
            jax.ShapeDtypeStruct((EROWS, 128), jnp.int32),
            jax.ShapeDtypeStruct((EROWS, 128), jnp.int32),
            jax.ShapeDtypeStruct((S_TBL,), jnp.float32),
        ),
        mesh=_MESH,
        compiler_params=pltpu.CompilerParams(needs_layout_passes=False),
        scratch_types=[
            pltpu.VMEM_SHARED((S_TBL,), jnp.int32),
            pltpu.VMEM_SHARED((S_TBL,), jnp.int32),
            pltpu.VMEM((_CR, 128), jnp.int32),
            pltpu.VMEM((_CR, 128), jnp.int32),
            pltpu.VMEM((_CR, 128), jnp.int32),
            pltpu.VMEM((_CR, 128), jnp.int32),
            pltpu.VMEM((_CR, 128), jnp.int32),
            pltpu.VMEM((KW_I,), jnp.float32),
            pltpu.SemaphoreType.DMA,
        ],
    )
    return f(src2, dst2, m_tbl, b_tbl, zeros_np)


def _sc_edge_weights_body(src_hbm, dst_hbm, dinv_hbm, ms2_hbm, dv2s_hbm,
                          zeros_hbm,
                          w_hbm, s2_hbm,
                          dinv_sh, sbuf, dbuf, va, vb, wbuf, mbuf, vbuf,
                          acc, sem):
    s = lax.axis_index("s")
    wid = _wid()

    @pl.when(s == 0)
    def _load():
        pltpu.sync_copy(dinv_hbm, dinv_sh)

    plsc.subcore_barrier()

    rowbase = wid * EW_ROWS

    def chunk1(ci, carry):
        ro = rowbase + ci * _CR
        pltpu.sync_copy(src_hbm.at[pl.ds(ro, _CR)], sbuf)
        pltpu.sync_copy(dst_hbm.at[pl.ds(ro, _CR)], dbuf)
        hs = []
        for j in range(_CR):
            hs.append(pltpu.async_copy(dinv_sh.at[sbuf.at[j]], va.at[j], sem))
            hs.append(pltpu.async_copy(dinv_sh.at[dbuf.at[j]], vb.at[j], sem))
        for h in hs:
            h.wait()
        for j in range(_CR):
            for l in range(8):
                sl = pl.ds(l * 16, 16)
                wbuf[j, sl] = va[j, sl] * vb[j, sl]
        pltpu.sync_copy(wbuf, w_hbm.at[pl.ds(ro, _CR)])
        return carry

    lax.fori_loop(0, EW_ROWS // _CR, chunk1, 0)

    kbase = wid * KW_S
    pltpu.sync_copy(zeros_hbm.at[pl.ds(0, KW_S)], acc)

    def chunk2(ci, carry):
        pltpu.sync_copy(ms2_hbm.at[pl.ds(ci * _CR, _CR)], mbuf)
        pltpu.sync_copy(dv2s_hbm.at[pl.ds(ci * _CR, _CR)], vbuf)
        for j in range(_CR):
            for l in range(8):
                sl = pl.ds(l * 16, 16)
                i16 = mbuf[j, sl]
                v16 = vbuf[j, sl]
                loc = i16 - kbase
                msk = (i16 >= kbase) & (i16 < kbase + KW_S)
                locc = jnp.minimum(jnp.maximum(loc, 0), KW_S - 1)
                plsc.addupdate_scatter(acc, [locc], v16, mask=msk)
        return carry

    lax.fori_loop(0, SROWS // _CR, chunk2, 0)
    pltpu.sync_copy(acc, s2_hbm.at[pl.ds(kbase, KW_S)])


def _sc_edge_weights(src2, dst2, dinv_tbl, m_s2, dv2_s2, zeros_np):
    f = pl.kernel(
        _sc_edge_weights_body,
        out_type=(
            jax.ShapeDtypeStruct((EROWS, 128), jnp.float32),
            jax.ShapeDtypeStruct((NP,), jnp.float32),
        ),
        mesh=_MESH,
        compiler_params=pltpu.CompilerParams(needs_layout_passes=False),
        scratch_types=[
            pltpu.VMEM_SHARED((S_TBL,), jnp.float32),
            pltpu.VMEM((_CR, 128), jnp.int32),
            pltpu.VMEM((_CR, 128), jnp.int32),
            pltpu.VMEM((_CR, 128), jnp.float32),
            pltpu.VMEM((_CR, 128), jnp.float32),
            pltpu.VMEM((_CR, 128), jnp.float32),
            pltpu.VMEM((_CR, 128), jnp.int32),
            pltpu.VMEM((_CR, 128), jnp.float32),
            pltpu.VMEM((KW_S,), jnp.float32),
            pltpu.SemaphoreType.DMA,
        ],
    )
    return f(src2, dst2, dinv_tbl, m_s2, dv2_s2, zeros_np)


def _sc_table_body(nrows, ki_hbm, kp_hbm, val_hbm, zeros_hbm, t_hbm,
                   acc, ib, pb, vb):
    wid = _wid()
    p0 = 2 * wid
    pltpu.sync_copy(zeros_hbm, acc.at[pl.ds(0, NP)])
    pltpu.sync_copy(zeros_hbm, acc.at[pl.ds(NP, NP)])

    def chunk(ci, carry):
        ro = ci * _CR
        pltpu.sync_copy(ki_hbm.at[pl.ds(ro, _CR)], ib)
        pltpu.sync_copy(kp_hbm.at[pl.ds(ro, _CR)], pb)
        pltpu.sync_copy(val_hbm.at[pl.ds(ro, _CR)], vb)
        for j in range(_CR):
            for l in range(8):
                sl = pl.ds(l * 16, 16)
                i16 = ib[j, sl]
                p16 = pb[j, sl]
                v16 = vb[j, sl]
                msk = (p16 >> 1) == wid
                idx = i16 + (p16 & 1) * NP
                plsc.addupdate_scatter(acc, [idx], v16, mask=msk)
        return carry

    lax.fori_loop(0, nrows // _CR, chunk, 0)
    pltpu.sync_copy(acc.at[pl.ds(0, NP)], t_hbm.at[pl.ds(p0 * NP, NP)])
    pltpu.sync_copy(acc.at[pl.ds(NP, NP)], t_hbm.at[pl.ds((p0 + 1) * NP, NP)])


def _sc_table(ki2, kp2, val2, nrows, zeros_np):
    f = pl.kernel(
        functools.partial(_sc_table_body, nrows),
        out_type=jax.ShapeDtypeStruct((P * NP,), jnp.float32),
        mesh=_MESH,
        compiler_params=pltpu.CompilerParams(needs_layout_passes=False),
        scratch_types=[
            pltpu.VMEM((2 * NP,), jnp.float32),
            pltpu.VMEM((_CR, 128), jnp.int32),
            pltpu.VMEM((_CR, 128), jnp.int32),
            pltpu.VMEM((_CR, 128), jnp.float32),
        ],
    )
    return f(ki2, kp2, val2, zeros_np).reshape(P, NP)


def _sc_kbuild_body(md_hbm, ms_hbm, w_hbm, mc_hbm, zeros_hbm, k_hbm,
                    mc_sh, mdb, msb, wb, idx0, g0, acc0, sem):
    c = lax.axis_index("c")
    s = lax.axis_index("s")

    for h in range(2):
        @pl.when(s == 0)
        def _load():
            pltpu.sync_copy(
                mc_hbm.at[pl.ds((c * 32 + h * 16) * NP, 16 * NP)], mc_sh)

        plsc.subcore_barrier()

        pltpu.sync_copy(zeros_hbm, acc0)
        off0 = s * NP
        p0 = c * 32 + h * 16 + s

        def chunk(ci, carry):
            ro = ci * _CR
            pltpu.sync_copy(md_hbm.at[pl.ds(ro, _CR)], mdb)
            pltpu.sync_copy(ms_hbm.at[pl.ds(ro, _CR)], msb)
            pltpu.sync_copy(w_hbm.at[pl.ds(ro, _CR)], wb)
            for j in range(_CR):
                for l in range(8):
                    sl = pl.ds(l * 16, 16)
                    idx0[j, sl] = mdb[j, sl] + off0
            hs = []
            for j in range(_CR):
                hs.append(
                    pltpu.async_copy(mc_sh.at[idx0.at[j]], g0.at[j], sem))
            for hh in hs:
                hh.wait()
            for j in range(_CR):
                for l in range(8):
                    sl = pl.ds(l * 16, 16)
                    i16 = msb[j, sl]
                    v16 = wb[j, sl]
                    plsc.addupdate_scatter(acc0, [i16], v16 * g0[j, sl])
            return carry

        lax.fori_loop(0, EROWS // _CR, chunk, 0)
        pltpu.sync_copy(acc0, k_hbm.at[pl.ds(p0 * NP, NP)])
        plsc.subcore_barrier()


def _sc_kbuild(md2, ms2, w2, mc_flat, zeros_np):
    f = pl.kernel(
        _sc_kbuild_body,
        out_type=jax.ShapeDtypeStruct((P * NP,), jnp.float32),
        mesh=_MESH,
        compiler_params=pltpu.CompilerParams(needs_layout_passes=False),
        scratch_types=[
            pltpu.VMEM_SHARED((16 * NP,), jnp.float32),
            pltpu.VMEM((_CR, 128), jnp.int32),
            pltpu.VMEM((_CR, 128), jnp.int32),
            pltpu.VMEM((_CR, 128), jnp.float32),
            pltpu.VMEM((_CR, 128), jnp.int32),
            pltpu.VMEM((_CR, 128), jnp.float32),
            pltpu.VMEM((NP,), jnp.float32),
            pltpu.SemaphoreType.DMA,
        ],
    )
    return f(md2, ms2, w2, mc_flat, zeros_np).reshape(P, NP)


def _tc_reduce_body(g_ref, bmat_ref, ct_ref, x_ref, s2x_ref,
                    w0_ref, b0_ref, wu_ref, bu_ref,
                    mc_ref, sg_ref, p1_ref, mcct_ref, smalls_ref, r_ref):
    i = pl.program_id(0)

    gblk = g_ref[...]
    bblk = bmat_ref[...]
    ctblk = ct_ref[...]
    xblk = x_ref[...]
    s2xblk = s2x_ref[...]

    mblk = gblk + bblk
    cm = jnp.maximum(jnp.sum(ctblk, axis=0, keepdims=True), 1.0)
    mcblk = mblk / cm
    mc_ref[...] = mcblk

    sg = jnp.dot(mblk, xblk, preferred_element_type=jnp.float32)
    p1 = jnp.dot(mcblk, s2xblk, preferred_element_type=jnp.float32)
    mcct = lax.dot_general(mcblk, ctblk, (((1,), (1,)), ((), ())),
                           preferred_element_type=jnp.float32)

    lane = lax.broadcasted_iota(jnp.int32, (P, D), 1)
    rs_m = jnp.sum(mblk, axis=1)
    rs_ct = jnp.sum(ctblk, axis=1)
    smalls = (jnp.where(lane == 0, rs_m[:, None], 0.0)
              + jnp.where(lane == 1, rs_ct[:, None], 0.0))

    @pl.when(i == 0)
    def _init():
        sg_ref[...] = sg
        p1_ref[...] = p1
        mcct_ref[...] = mcct
        smalls_ref[...] = smalls

    @pl.when(i > 0)
    def _acc():
        sg_ref[...] += sg
        p1_ref[...] += p1
        mcct_ref[...] += mcct
        smalls_ref[...] += smalls

    @pl.when(i == GRID - 1)
    def _epilogue():
        sm = smalls_ref[...]
        cnt_b = jnp.sum(jnp.where(lane == 1, sm, 0.0), axis=1)
        cb = jnp.maximum(cnt_b, 1.0)
        sgpool = (jnp.dot(sg_ref[...] / cb[:, None], w0_ref[...],
                          preferred_element_type=jnp.float32)
                  + b0_ref[...])
        r_ref[...] = jax.nn.relu(
            jnp.dot(sgpool, wu_ref[...], preferred_element_type=jnp.float32)
            + bu_ref[...])


def _tc_reduce(G, B, Ct, x, s2x, W0, b0, Wu, bu):
    tbl = pl.BlockSpec((P, BN), lambda i: (0, i))
    xsp = pl.BlockSpec((BN, D), lambda i: (i, 0))
    wsp = pl.BlockSpec((D, D), lambda i: (0, 0))
    bsp = pl.BlockSpec((1, D), lambda i: (0, 0))
    acc = pl.BlockSpec((P, D), lambda i: (0, 0))
    out_shapes = (
        jax.ShapeDtypeStruct((P, NP), jnp.float32),
        jax.ShapeDtypeStruct((P, D), jnp.float32),
        jax.ShapeDtypeStruct((P, D), jnp.float32),
        jax.ShapeDtypeStruct((P, P), jnp.float32),
        jax.ShapeDtypeStruct((P, D), jnp.float32),
        jax.ShapeDtypeStruct((P, D), jnp.float32),
    )
    return pl.pallas_call(
        _tc_reduce_body,
        grid=(GRID,),
        in_specs=[tbl, tbl, tbl, xsp, xsp, wsp, bsp, wsp, bsp],
        out_specs=(tbl, acc, acc, pl.BlockSpec((P, P), lambda i: (0, 0)),
                   acc, acc),
        out_shape=out_shapes,
        compiler_params=pltpu.CompilerParams(
            dimension_semantics=("arbitrary",)),
    )(G, B, Ct, x, s2x, W0, b0, Wu, bu)


def _tc_out_body(k_ref, x_ref, p1_ref, mcct_ref, r_ref, smalls_ref,
                 w0_ref, b0_ref, w1_ref, b1_ref,
                 kx_ref, out_ref):
    i = pl.program_id(0)
    kx = jnp.dot(k_ref[...], x_ref[...], preferred_element_type=jnp.float32)

    @pl.when(i == 0)
    def _init():
        kx_ref[...] = kx

    @pl.when(i > 0)
    def _acc():
        kx_ref[...] += kx

    @pl.when(i == GRID - 1)
    def _epilogue():
        lane = lax.broadcasted_iota(jnp.int32, (P, D), 1)
        sm = smalls_ref[...]
        colsum_m = jnp.sum(jnp.where(lane == 0, sm, 0.0), axis=1)
        cnt_b = jnp.sum(jnp.where(lane == 1, sm, 0.0), axis=1)
        cb = jnp.maximum(cnt_b, 1.0)
        out_sum = (jnp.dot(kx_ref[...] + p1_ref[...], w0_ref[...],
                           preferred_element_type=jnp.float32)
                   + jnp.dot(mcct_ref[...], r_ref[...],
                             preferred_element_type=jnp.float32)
                   + colsum_m[:, None] * b0_ref[...])
        out = (jnp.dot(out_sum / cb[:, None], w1_ref[...],
                       preferred_element_type=jnp.float32)
               + b1_ref[...])
        out_ref[...] = jnp.where(cnt_b[:, None] > 0, out, 0.0)


def _tc_out(K, x, partial1, McCt, r, smalls, W0, b0, W1, b1):
    tbl = pl.BlockSpec((P, BN), lambda i: (0, i))
    xsp = pl.BlockSpec((BN, D), lambda i: (i, 0))
    wsp = pl.BlockSpec((D, D), lambda i: (0, 0))
    bsp = pl.BlockSpec((1, D), lambda i: (0, 0))
    acc = pl.BlockSpec((P, D), lambda i: (0, 0))
    out_shapes = (
        jax.ShapeDtypeStruct((P, D), jnp.float32),
        jax.ShapeDtypeStruct((P, D), jnp.float32),
    )
    res = pl.pallas_call(
        _tc_out_body,
        grid=(GRID,),
        in_specs=[tbl, xsp, acc, pl.BlockSpec((P, P), lambda i: (0, 0)),
                  acc, acc, wsp, bsp, wsp, bsp],
        out_specs=(acc, acc),
        out_shape=out_shapes,
        compiler_params=pltpu.CompilerParams(
            dimension_semantics=("arbitrary",)),
    )(K, x, partial1, McCt, r, smalls, W0, b0, W1, b1)
    return res[1]


def kernel(x, edge_attr, subgraphs_nodes_mapper, subgraphs_edges_mapper,
           combined_subgraphs, subgraphs_batch, W0, b0, W1, b1, Wu, bu):
    m = subgraphs_nodes_mapper
    b = subgraphs_batch
    src = combined_subgraphs[0]
    dst = combined_subgraphs[1]

    zeros_np = jnp.zeros((NP,), jnp.float32)

    src2 = jnp.pad(src, (0, E_PAD - E_SUB),
                   constant_values=N_SUB).reshape(EROWS, 128)
    dst2 = jnp.pad(dst, (0, E_PAD - E_SUB),
                   constant_values=N_SUB).reshape(EROWS, 128)
    m_tbl = jnp.pad(m, (0, S_TBL - N_SUB), constant_values=NP - 1)
    b_tbl = jnp.pad(b, (0, S_TBL - N_SUB), constant_values=-1)

    ms2, md2, bd2, indeg = _sc_gather_maps(src2, dst2, m_tbl, b_tbl, zeros_np)

    deg = 1.0 + indeg[:N_SUB]
    dinv = lax.rsqrt(deg)
    dinv2 = 1.0 / deg
    dinv_tbl = jnp.pad(dinv, (0, S_TBL - N_SUB))

    m_s2 = jnp.pad(m, (0, S_STREAM - N_SUB),
                   constant_values=-1).reshape(SROWS, 128)
    b_s2 = jnp.pad(b, (0, S_STREAM - N_SUB),
                   constant_values=-1).reshape(SROWS, 128)
    dv2_s2 = jnp.pad(dinv2, (0, S_STREAM - N_SUB)).reshape(SROWS, 128)
    ones_s2 = jnp.ones((SROWS, 128), jnp.float32)

    w2, s2 = _sc_edge_weights(src2, dst2, dinv_tbl, m_s2, dv2_s2, zeros_np)

    G = _sc_table(ms2, bd2, w2, EROWS, zeros_np)
    Bm = _sc_table(m_s2, b_s2, dv2_s2, SROWS, zeros_np)
    Ct = _sc_table(m_s2, b_s2, ones_s2, SROWS, zeros_np)

    xp = jnp.pad(x, ((0, NP - N_NODES), (0, 0)))
    s2x = xp * s2[:, None]

    Mc, sg_sum, partial1, McCt, smalls, r = _tc_reduce(
        G, Bm, Ct, xp, s2x, W0, b0[None, :], Wu, bu[None, :])

    K = _sc_kbuild(md2, ms2, w2, Mc.reshape(P * NP), zeros_np)

    return _tc_out(K, xp, partial1, McCt, r, smalls, W0, b0[None, :],
                   W1, b1[None, :])


# speedup vs baseline: 22.2890x; 1.1383x over previous
import functools
import jax
import jax.numpy as jnp
from jax import lax
from jax.experimental import pallas as pl
from jax.experimental.pallas import tpu as pltpu
from jax.experimental.pallas import tpu_sc as plsc

N_NODES = 50000
N_SUB = 150000
E_SUB = 480000
D = 128
P = 64

NP = 51200
BN = 1024
GRID = NP // BN

E_PAD = 491520
EROWS = E_PAD // 128
EW_ROWS = EROWS // 32
S_TBL = 150016
S_STREAM = 150528
SROWS = S_STREAM // 128
KW_I = S_TBL // 32
KW_S = NP // 32
_CR = 24

_MESH = plsc.VectorSubcoreMesh(core_axis_name="c", subcore_axis_name="s",
                               num_cores=2, num_subcores=16)


def _wid():
    return lax.axis_index("c") * 16 + lax.axis_index("s")


def _sc_gather_maps_body(src_hbm, dst_hbm, mtbl_hbm, btbl_hbm, zeros_hbm,
                         ms_hbm, md_hbm, bd_hbm, indeg_hbm,
                         m_sh, b_sh, sbuf, dbuf, msv, mdv, bdv, acc, sem):
    s = lax.axis_index("s")
    wid = _wid()

    @pl.when(s == 0)
    def _load():
        pltpu.sync_copy(mtbl_hbm, m_sh)
        pltpu.sync_copy(btbl_hbm, b_sh)

    plsc.subcore_barrier()

    rowbase = wid * EW_ROWS

    def chunk1(ci, carry):
        ro = rowbase + ci * _CR
        pltpu.sync_copy(src_hbm.at[pl.ds(ro, _CR)], sbuf)
        pltpu.sync_copy(dst_hbm.at[pl.ds(ro, _CR)], dbuf)
        hs = []
        for j in range(_CR):
            hs.append(pltpu.async_copy(m_sh.at[sbuf.at[j]], msv.at[j], sem))
            hs.append(pltpu.async_copy(m_sh.at[dbuf.at[j]], mdv.at[j], sem))
            hs.append(pltpu.async_copy(b_sh.at[dbuf.at[j]], bdv.at[j], sem))
        for h in hs:
            h.wait()
        pltpu.sync_copy(msv, ms_hbm.at[pl.ds(ro, _CR)])
        pltpu.sync_copy(mdv, md_hbm.at[pl.ds(ro, _CR)])
        pltpu.sync_copy(bdv, bd_hbm.at[pl.ds(ro, _CR)])
        return carry

    lax.fori_loop(0, EW_ROWS // _CR, chunk1, 0)

    kbase = wid * KW_I
    pltpu.sync_copy(zeros_hbm.at[pl.ds(0, KW_I)], acc)
    ones16 = jnp.ones((16,), jnp.float32)

    def chunk2(ci, carry):
        pltpu.sync_copy(dst_hbm.at[pl.ds(ci * _CR, _CR)], dbuf)
        for j in range(_CR):
            for l in range(8):
                d16 = dbuf[j, pl.ds(l * 16, 16)]
                loc = d16 - kbase
                msk = (d16 >= kbase) & (d16 < kbase + KW_I)
                locc = jnp.minimum(jnp.maximum(loc, 0), KW_I - 1)
                plsc.addupdate_scatter(acc, [locc], ones16, mask=msk)
        return carry

    lax.fori_loop(0, EROWS // _CR, chunk2, 0)
    pltpu.sync_copy(acc, indeg_hbm.at[pl.ds(kbase, KW_I)])


def _sc_gather_maps(src2, dst2, m_tbl, b_tbl, zeros_np):
    f = pl.kernel(
        _sc_gather_maps_body,
        out_type=(
            jax.ShapeDtypeStruct((EROWS, 128), jnp.int32),
            jax.ShapeDtypeStruct((EROWS, 128), jnp.int32),
            jax.ShapeDtypeStruct((EROWS, 128), jnp.int32),
            jax.ShapeDtypeStruct((S_TBL,), jnp.float32),
        ),
        mesh=_MESH,
        compiler_params=pltpu.CompilerParams(needs_layout_passes=False),
        scratch_types=[
            pltpu.VMEM_SHARED((S_TBL,), jnp.int32),
            pltpu.VMEM_SHARED((S_TBL,), jnp.int32),
            pltpu.VMEM((_CR, 128), jnp.int32),
            pltpu.VMEM((_CR, 128), jnp.int32),
            pltpu.VMEM((_CR, 128), jnp.int32),
            pltpu.VMEM((_CR, 128), jnp.int32),
            pltpu.VMEM((_CR, 128), jnp.int32),
            pltpu.VMEM((KW_I,), jnp.float32),
            pltpu.SemaphoreType.DMA,
        ],
    )
    return f(src2, dst2, m_tbl, b_tbl, zeros_np)


def _sc_edge_weights_body(src_hbm, dst_hbm, dinv_hbm, ms2_hbm, dv2s_hbm,
                          zeros_hbm,
                          w_hbm, s2_hbm,
                          dinv_sh, sbuf, dbuf, va, vb, wbuf, mbuf, vbuf,
                          acc, sem):
    s = lax.axis_index("s")
    wid = _wid()

    @pl.when(s == 0)
    def _load():
        pltpu.sync_copy(dinv_hbm, dinv_sh)

    plsc.subcore_barrier()

    rowbase = wid * EW_ROWS

    def chunk1(ci, carry):
        ro = rowbase + ci * _CR
        pltpu.sync_copy(src_hbm.at[pl.ds(ro, _CR)], sbuf)
        pltpu.sync_copy(dst_hbm.at[pl.ds(ro, _CR)], dbuf)
        hs = []
        for j in range(_CR):
            hs.append(pltpu.async_copy(dinv_sh.at[sbuf.at[j]], va.at[j], sem))
            hs.append(pltpu.async_copy(dinv_sh.at[dbuf.at[j]], vb.at[j], sem))
        for h in hs:
            h.wait()
        for j in range(_CR):
            for l in range(8):
                sl = pl.ds(l * 16, 16)
                wbuf[j, sl] = va[j, sl] * vb[j, sl]
        pltpu.sync_copy(wbuf, w_hbm.at[pl.ds(ro, _CR)])
        return carry

    lax.fori_loop(0, EW_ROWS // _CR, chunk1, 0)

    kbase = wid * KW_S
    pltpu.sync_copy(zeros_hbm.at[pl.ds(0, KW_S)], acc)

    def chunk2(ci, carry):
        pltpu.sync_copy(ms2_hbm.at[pl.ds(ci * _CR, _CR)], mbuf)
        pltpu.sync_copy(dv2s_hbm.at[pl.ds(ci * _CR, _CR)], vbuf)
        for j in range(_CR):
            for l in range(8):
                sl = pl.ds(l * 16, 16)
                i16 = mbuf[j, sl]
                v16 = vbuf[j, sl]
                loc = i16 - kbase
                msk = (i16 >= kbase) & (i16 < kbase + KW_S)
                locc = jnp.minimum(jnp.maximum(loc, 0), KW_S - 1)
                plsc.addupdate_scatter(acc, [locc], v16, mask=msk)
        return carry

    lax.fori_loop(0, SROWS // _CR, chunk2, 0)
    pltpu.sync_copy(acc, s2_hbm.at[pl.ds(kbase, KW_S)])


def _sc_edge_weights(src2, dst2, dinv_tbl, m_s2, dv2_s2, zeros_np):
    f = pl.kernel(
        _sc_edge_weights_body,
        out_type=(
            jax.ShapeDtypeStruct((EROWS, 128), jnp.float32),
            jax.ShapeDtypeStruct((NP,), jnp.float32),
        ),
        mesh=_MESH,
        compiler_params=pltpu.CompilerParams(needs_layout_passes=False),
        scratch_types=[
            pltpu.VMEM_SHARED((S_TBL,), jnp.float32),
            pltpu.VMEM((_CR, 128), jnp.int32),
            pltpu.VMEM((_CR, 128), jnp.int32),
            pltpu.VMEM((_CR, 128), jnp.float32),
            pltpu.VMEM((_CR, 128), jnp.float32),
            pltpu.VMEM((_CR, 128), jnp.float32),
            pltpu.VMEM((_CR, 128), jnp.int32),
            pltpu.VMEM((_CR, 128), jnp.float32),
            pltpu.VMEM((KW_S,), jnp.float32),
            pltpu.SemaphoreType.DMA,
        ],
    )
    return f(src2, dst2, dinv_tbl, m_s2, dv2_s2, zeros_np)


def _sc_table_body(nrows, ki_hbm, kp_hbm, val_hbm, zeros_hbm, t_hbm,
                   acc, ib, pb, vb):
    wid = _wid()
    p0 = 2 * wid
    pltpu.sync_copy(zeros_hbm, acc.at[pl.ds(0, NP)])
    pltpu.sync_copy(zeros_hbm, acc.at[pl.ds(NP, NP)])

    def chunk(ci, carry):
        ro = ci * _CR
        pltpu.sync_copy(ki_hbm.at[pl.ds(ro, _CR)], ib)
        pltpu.sync_copy(kp_hbm.at[pl.ds(ro, _CR)], pb)
        pltpu.sync_copy(val_hbm.at[pl.ds(ro, _CR)], vb)
        for j in range(_CR):
            for l in range(8):
                sl = pl.ds(l * 16, 16)
                i16 = ib[j, sl]
                p16 = pb[j, sl]
                v16 = vb[j, sl]
                msk = (p16 >> 1) == wid
                idx = i16 + (p16 & 1) * NP
                plsc.addupdate_scatter(acc, [idx], v16, mask=msk)
        return carry

    lax.fori_loop(0, nrows // _CR, chunk, 0)
    pltpu.sync_copy(acc.at[pl.ds(0, NP)], t_hbm.at[pl.ds(p0 * NP, NP)])
    pltpu.sync_copy(acc.at[pl.ds(NP, NP)], t_hbm.at[pl.ds((p0 + 1) * NP, NP)])


def _sc_table(ki2, kp2, val2, nrows, zeros_np):
    f = pl.kernel(
        functools.partial(_sc_table_body, nrows),
        out_type=jax.ShapeDtypeStruct((P * NP,), jnp.float32),
        mesh=_MESH,
        compiler_params=pltpu.CompilerParams(needs_layout_passes=False),
        scratch_types=[
            pltpu.VMEM((2 * NP,), jnp.float32),
            pltpu.VMEM((_CR, 128), jnp.int32),
            pltpu.VMEM((_CR, 128), jnp.int32),
            pltpu.VMEM((_CR, 128), jnp.float32),
        ],
    )
    return f(ki2, kp2, val2, zeros_np).reshape(P, NP)


_CRK = 16
_NCHK = EROWS // _CRK


def _sc_kbuild_body(md_hbm, ms_hbm, w_hbm, mc_hbm, zeros_hbm, k_hbm,
                    mc_sh, mdb, msb, wb, idx, g, acc0, semS, semG):
    c = lax.axis_index("c")
    s = lax.axis_index("s")

    def stage(i, p):
        ro = i * _CRK
        pltpu.async_copy(md_hbm.at[pl.ds(ro, _CRK)], mdb.at[p], semS)
        pltpu.async_copy(ms_hbm.at[pl.ds(ro, _CRK)], msb.at[p], semS)
        pltpu.async_copy(w_hbm.at[pl.ds(ro, _CRK)], wb.at[p], semS)

    def drain_stage(i, p):
        ro = i * _CRK
        pltpu.make_async_copy(md_hbm.at[pl.ds(ro, _CRK)], mdb.at[p], semS).wait()
        pltpu.make_async_copy(ms_hbm.at[pl.ds(ro, _CRK)], msb.at[p], semS).wait()
        pltpu.make_async_copy(w_hbm.at[pl.ds(ro, _CRK)], wb.at[p], semS).wait()

    for h in range(2):
        @pl.when(s == 0)
        def _load():
            pltpu.sync_copy(
                mc_hbm.at[pl.ds((c * 32 + h * 16) * NP, 16 * NP)], mc_sh)

        plsc.subcore_barrier()

        pltpu.sync_copy(zeros_hbm, acc0)
        off0 = s * NP
        p0 = c * 32 + h * 16 + s

        stage(0, 0)

        def body_one(i, p):
            drain_stage(i, p)
            for j in range(_CRK):
                for l in range(8):
                    sl = pl.ds(l * 16, 16)
                    idx[p, j, sl] = mdb[p, j, sl] + off0
            for j in range(_CRK):
                pltpu.async_copy(mc_sh.at[idx.at[p, j]], g.at[p, j], semG)

            @pl.when(i > 0)
            def _():
                q = 1 - p
                for j in range(_CRK):
                    pltpu.make_async_copy(mc_sh.at[idx.at[q, j]],
                                          g.at[q, j], semG).wait()
                for j in range(_CRK):
                    for l in range(8):
                        sl = pl.ds(l * 16, 16)
                        plsc.addupdate_scatter(
                            acc0, [msb[q, j, sl]], wb[q, j, sl] * g[q, j, sl])

            @pl.when(i + 1 < _NCHK)
            def _():
                stage(i + 1, 1 - p)

        def pair(k, carry):
            body_one(2 * k, 0)
            body_one(2 * k + 1, 1)
            return carry

        lax.fori_loop(0, _NCHK // 2, pair, 0)

        qe = 1
        for j in range(_CRK):
            pltpu.make_async_copy(mc_sh.at[idx.at[qe, j]],
                                  g.at[qe, j], semG).wait()
        for j in range(_CRK):
            for l in range(8):
                sl = pl.ds(l * 16, 16)
                plsc.addupdate_scatter(
                    acc0, [msb[qe, j, sl]], wb[qe, j, sl] * g[qe, j, sl])

        pltpu.sync_copy(acc0, k_hbm.at[pl.ds(p0 * NP, NP)])
        plsc.subcore_barrier()


def _sc_kbuild(md2, ms2, w2, mc_flat, zeros_np):
    f = pl.kernel(
        _sc_kbuild_body,
        out_type=jax.ShapeDtypeStruct((P * NP,), jnp.float32),
        mesh=_MESH,
        compiler_params=pltpu.CompilerParams(needs_layout_passes=False),
        scratch_types=[
            pltpu.VMEM_SHARED((16 * NP,), jnp.float32),
            pltpu.VMEM((2, _CRK, 128), jnp.int32),
            pltpu.VMEM((2, _CRK, 128), jnp.int32),
            pltpu.VMEM((2, _CRK, 128), jnp.float32),
            pltpu.VMEM((2, _CRK, 128), jnp.int32),
            pltpu.VMEM((2, _CRK, 128), jnp.float32),
            pltpu.VMEM((NP,), jnp.float32),
            pltpu.SemaphoreType.DMA,
            pltpu.SemaphoreType.DMA,
        ],
    )
    return f(md2, ms2, w2, mc_flat, zeros_np).reshape(P, NP)


def _tc_reduce_body(g_ref, bmat_ref, ct_ref, x_ref, s2x_ref,
                    w0_ref, b0_ref, wu_ref, bu_ref,
                    mc_ref, sg_ref, p1_ref, mcct_ref, smalls_ref, r_ref):
    i = pl.program_id(0)

    gblk = g_ref[...]
    bblk = bmat_ref[...]
    ctblk = ct_ref[...]
    xblk = x_ref[...]
    s2xblk = s2x_ref[...]

    mblk = gblk + bblk
    cm = jnp.maximum(jnp.sum(ctblk, axis=0, keepdims=True), 1.0)
    mcblk = mblk / cm
    mc_ref[...] = mcblk

    sg = jnp.dot(mblk, xblk, preferred_element_type=jnp.float32)
    p1 = jnp.dot(mcblk, s2xblk, preferred_element_type=jnp.float32)
    mcct = lax.dot_general(mcblk, ctblk, (((1,), (1,)), ((), ())),
                           preferred_element_type=jnp.float32)

    lane = lax.broadcasted_iota(jnp.int32, (P, D), 1)
    rs_m = jnp.sum(mblk, axis=1)
    rs_ct = jnp.sum(ctblk, axis=1)
    smalls = (jnp.where(lane == 0, rs_m[:, None], 0.0)
              + jnp.where(lane == 1, rs_ct[:, None], 0.0))

    @pl.when(i == 0)
    def _init():
        sg_ref[...] = sg
        p1_ref[...] = p1
        mcct_ref[...] = mcct
        smalls_ref[...] = smalls

    @pl.when(i > 0)
    def _acc():
        sg_ref[...] += sg
        p1_ref[...] += p1
        mcct_ref[...] += mcct
        smalls_ref[...] += smalls

    @pl.when(i == GRID - 1)
    def _epilogue():
        sm = smalls_ref[...]
        cnt_b = jnp.sum(jnp.where(lane == 1, sm, 0.0), axis=1)
        cb = jnp.maximum(cnt_b, 1.0)
        sgpool = (jnp.dot(sg_ref[...] / cb[:, None], w0_ref[...],
                          preferred_element_type=jnp.float32)
                  + b0_ref[...])
        r_ref[...] = jax.nn.relu(
            jnp.dot(sgpool, wu_ref[...], preferred_element_type=jnp.float32)
            + bu_ref[...])


def _tc_reduce(G, B, Ct, x, s2x, W0, b0, Wu, bu):
    tbl = pl.BlockSpec((P, BN), lambda i: (0, i))
    xsp = pl.BlockSpec((BN, D), lambda i: (i, 0))
    wsp = pl.BlockSpec((D, D), lambda i: (0, 0))
    bsp = pl.BlockSpec((1, D), lambda i: (0, 0))
    acc = pl.BlockSpec((P, D), lambda i: (0, 0))
    out_shapes = (
        jax.ShapeDtypeStruct((P, NP), jnp.float32),
        jax.ShapeDtypeStruct((P, D), jnp.float32),
        jax.ShapeDtypeStruct((P, D), jnp.float32),
        jax.ShapeDtypeStruct((P, P), jnp.float32),
        jax.ShapeDtypeStruct((P, D), jnp.float32),
        jax.ShapeDtypeStruct((P, D), jnp.float32),
    )
    return pl.pallas_call(
        _tc_reduce_body,
        grid=(GRID,),
        in_specs=[tbl, tbl, tbl, xsp, xsp, wsp, bsp, wsp, bsp],
        out_specs=(tbl, acc, acc, pl.BlockSpec((P, P), lambda i: (0, 0)),
                   acc, acc),
        out_shape=out_shapes,
        compiler_params=pltpu.CompilerParams(
            dimension_semantics=("arbitrary",)),
    )(G, B, Ct, x, s2x, W0, b0, Wu, bu)


def _tc_out_body(k_ref, x_ref, p1_ref, mcct_ref, r_ref, smalls_ref,
                 w0_ref, b0_ref, w1_ref, b1_ref,
                 kx_ref, out_ref):
    i = pl.program_id(0)
    kx = jnp.dot(k_ref[...], x_ref[...], preferred_element_type=jnp.float32)

    @pl.when(i == 0)
    def _init():
        kx_ref[...] = kx

    @pl.when(i > 0)
    def _acc():
        kx_ref[...] += kx

    @pl.when(i == GRID - 1)
    def _epilogue():
        lane = lax.broadcasted_iota(jnp.int32, (P, D), 1)
        sm = smalls_ref[...]
        colsum_m = jnp.sum(jnp.where(lane == 0, sm, 0.0), axis=1)
        cnt_b = jnp.sum(jnp.where(lane == 1, sm, 0.0), axis=1)
        cb = jnp.maximum(cnt_b, 1.0)
        out_sum = (jnp.dot(kx_ref[...] + p1_ref[...], w0_ref[...],
                           preferred_element_type=jnp.float32)
                   + jnp.dot(mcct_ref[...], r_ref[...],
                             preferred_element_type=jnp.float32)
                   + colsum_m[:, None] * b0_ref[...])
        out = (jnp.dot(out_sum / cb[:, None], w1_ref[...],
                       preferred_element_type=jnp.float32)
               + b1_ref[...])
        out_ref[...] = jnp.where(cnt_b[:, None] > 0, out, 0.0)


def _tc_out(K, x, partial1, McCt, r, smalls, W0, b0, W1, b1):
    tbl = pl.BlockSpec((P, BN), lambda i: (0, i))
    xsp = pl.BlockSpec((BN, D), lambda i: (i, 0))
    wsp = pl.BlockSpec((D, D), lambda i: (0, 0))
    bsp = pl.BlockSpec((1, D), lambda i: (0, 0))
    acc = pl.BlockSpec((P, D), lambda i: (0, 0))
    out_shapes = (
        jax.ShapeDtypeStruct((P, D), jnp.float32),
        jax.ShapeDtypeStruct((P, D), jnp.float32),
    )
    res = pl.pallas_call(
        _tc_out_body,
        grid=(GRID,),
        in_specs=[tbl, xsp, acc, pl.BlockSpec((P, P), lambda i: (0, 0)),
                  acc, acc, wsp, bsp, wsp, bsp],
        out_specs=(acc, acc),
        out_shape=out_shapes,
        compiler_params=pltpu.CompilerParams(
            dimension_semantics=("arbitrary",)),
    )(K, x, partial1, McCt, r, smalls, W0, b0, W1, b1)
    return res[1]


def kernel(x, edge_attr, subgraphs_nodes_mapper, subgraphs_edges_mapper,
           combined_subgraphs, subgraphs_batch, W0, b0, W1, b1, Wu, bu):
    m = subgraphs_nodes_mapper
    b = subgraphs_batch
    src = combined_subgraphs[0]
    dst = combined_subgraphs[1]

    zeros_np = jnp.zeros((NP,), jnp.float32)

    src2 = jnp.pad(src, (0, E_PAD - E_SUB),
                   constant_values=N_SUB).reshape(EROWS, 128)
    dst2 = jnp.pad(dst, (0, E_PAD - E_SUB),
                   constant_values=N_SUB).reshape(EROWS, 128)
    m_tbl = jnp.pad(m, (0, S_TBL - N_SUB), constant_values=NP - 1)
    b_tbl = jnp.pad(b, (0, S_TBL - N_SUB), constant_values=-1)

    ms2, md2, bd2, indeg = _sc_gather_maps(src2, dst2, m_tbl, b_tbl, zeros_np)

    deg = 1.0 + indeg[:N_SUB]
    dinv = lax.rsqrt(deg)
    dinv2 = 1.0 / deg
    dinv_tbl = jnp.pad(dinv, (0, S_TBL - N_SUB))

    m_s2 = jnp.pad(m, (0, S_STREAM - N_SUB),
                   constant_values=-1).reshape(SROWS, 128)
    b_s2 = jnp.pad(b, (0, S_STREAM - N_SUB),
                   constant_values=-1).reshape(SROWS, 128)
    dv2_s2 = jnp.pad(dinv2, (0, S_STREAM - N_SUB)).reshape(SROWS, 128)
    ones_s2 = jnp.ones((SROWS, 128), jnp.float32)

    w2, s2 = _sc_edge_weights(src2, dst2, dinv_tbl, m_s2, dv2_s2, zeros_np)

    G = _sc_table(ms2, bd2, w2, EROWS, zeros_np)
    Bm = _sc_table(m_s2, b_s2, dv2_s2, SROWS, zeros_np)
    Ct = _sc_table(m_s2, b_s2, ones_s2, SROWS, zeros_np)

    xp = jnp.pad(x, ((0, NP - N_NODES), (0, 0)))
    s2x = xp * s2[:, None]

    Mc, sg_sum, partial1, McCt, smalls, r = _tc_reduce(
        G, Bm, Ct, xp, s2x, W0, b0[None, :], Wu, bu[None, :])

    K = _sc_kbuild(md2, ms2, w2, Mc.reshape(P * NP), zeros_np)

    return _tc_out(K, xp, partial1, McCt, r, smalls, W0, b0[None, :],
                   W1, b1[None, :])


# speedup vs baseline: 25.1660x; 1.1291x over previous
import functools
import jax
import jax.numpy as jnp
from jax import lax
from jax.experimental import pallas as pl
from jax.experimental.pallas import tpu as pltpu
from jax.experimental.pallas import tpu_sc as plsc

N_NODES = 50000
N_SUB = 150000
E_SUB = 480000
D = 128
P = 64

NP = 51200
BN = 1024
GRID = NP // BN

E_PAD = 491520
EROWS = E_PAD // 128
EW_ROWS = EROWS // 32
S_TBL = 150016
S_STREAM = 150528
SROWS = S_STREAM // 128
KW_I = S_TBL // 32
KW_S = NP // 32
_CR = 24

_MESH = plsc.VectorSubcoreMesh(core_axis_name="c", subcore_axis_name="s",
                               num_cores=2, num_subcores=16)


def _wid():
    return lax.axis_index("c") * 16 + lax.axis_index("s")


def _sc_gather_maps_body(src_hbm, dst_hbm, mtbl_hbm, btbl_hbm, zeros_hbm,
                         ms_hbm, md_hbm, bd_hbm, indeg_hbm,
                         m_sh, b_sh, sbuf, dbuf, msv, mdv, bdv, acc, sem):
    s = lax.axis_index("s")
    wid = _wid()

    @pl.when(s == 0)
    def _load():
        pltpu.sync_copy(mtbl_hbm, m_sh)
        pltpu.sync_copy(btbl_hbm, b_sh)

    plsc.subcore_barrier()

    rowbase = wid * EW_ROWS

    def chunk1(ci, carry):
        ro = rowbase + ci * _CR
        pltpu.sync_copy(src_hbm.at[pl.ds(ro, _CR)], sbuf)
        pltpu.sync_copy(dst_hbm.at[pl.ds(ro, _CR)], dbuf)
        hs = []
        for j in range(_CR):
            hs.append(pltpu.async_copy(m_sh.at[sbuf.at[j]], msv.at[j], sem))
            hs.append(pltpu.async_copy(m_sh.at[dbuf.at[j]], mdv.at[j], sem))
            hs.append(pltpu.async_copy(b_sh.at[dbuf.at[j]], bdv.at[j], sem))
        for h in hs:
            h.wait()
        pltpu.sync_copy(msv, ms_hbm.at[pl.ds(ro, _CR)])
        pltpu.sync_copy(mdv, md_hbm.at[pl.ds(ro, _CR)])
        pltpu.sync_copy(bdv, bd_hbm.at[pl.ds(ro, _CR)])
        return carry

    lax.fori_loop(0, EW_ROWS // _CR, chunk1, 0)

    kbase = wid * KW_I
    pltpu.sync_copy(zeros_hbm.at[pl.ds(0, KW_I)], acc)
    ones16 = jnp.ones((16,), jnp.float32)

    def chunk2(ci, carry):
        pltpu.sync_copy(dst_hbm.at[pl.ds(ci * _CR, _CR)], dbuf)
        for j in range(_CR):
            for l in range(8):
                d16 = dbuf[j, pl.ds(l * 16, 16)]
                loc = d16 - kbase
                msk = (d16 >= kbase) & (d16 < kbase + KW_I)
                locc = jnp.minimum(jnp.maximum(loc, 0), KW_I - 1)
                plsc.addupdate_scatter(acc, [locc], ones16, mask=msk)
        return carry

    lax.fori_loop(0, EROWS // _CR, chunk2, 0)
    pltpu.sync_copy(acc, indeg_hbm.at[pl.ds(kbase, KW_I)])


def _sc_gather_maps(src2, dst2, m_tbl, b_tbl, zeros_np):
    f = pl.kernel(
        _sc_gather_maps_body,
        out_type=(
            jax.ShapeDtypeStruct((EROWS, 128), jnp.int32),
            jax.ShapeDtypeStruct((EROWS, 128), jnp.int32),
            jax.ShapeDtypeStruct((EROWS, 128), jnp.int32),
            jax.ShapeDtypeStruct((S_TBL,), jnp.float32),
        ),
        mesh=_MESH,
        compiler_params=pltpu.CompilerParams(needs_layout_passes=False),
        scratch_types=[
            pltpu.VMEM_SHARED((S_TBL,), jnp.int32),
            pltpu.VMEM_SHARED((S_TBL,), jnp.int32),
            pltpu.VMEM((_CR, 128), jnp.int32),
            pltpu.VMEM((_CR, 128), jnp.int32),
            pltpu.VMEM((_CR, 128), jnp.int32),
            pltpu.VMEM((_CR, 128), jnp.int32),
            pltpu.VMEM((_CR, 128), jnp.int32),
            pltpu.VMEM((KW_I,), jnp.float32),
            pltpu.SemaphoreType.DMA,
        ],
    )
    return f(src2, dst2, m_tbl, b_tbl, zeros_np)


def _sc_edge_weights_body(src_hbm, dst_hbm, dinv_hbm, ms2_hbm, dv2s_hbm,
                          zeros_hbm,
                          w_hbm, s2_hbm,
                          dinv_sh, sbuf, dbuf, va, vb, wbuf, mbuf, vbuf,
                          acc, sem):
    s = lax.axis_index("s")
    wid = _wid()

    @pl.when(s == 0)
    def _load():
        pltpu.sync_copy(dinv_hbm, dinv_sh)

    plsc.subcore_barrier()

    rowbase = wid * EW_ROWS

    def chunk1(ci, carry):
        ro = rowbase + ci * _CR
        pltpu.sync_copy(src_hbm.at[pl.ds(ro, _CR)], sbuf)
        pltpu.sync_copy(dst_hbm.at[pl.ds(ro, _CR)], dbuf)
        hs = []
        for j in range(_CR):
            hs.append(pltpu.async_copy(dinv_sh.at[sbuf.at[j]], va.at[j], sem))
            hs.append(pltpu.async_copy(dinv_sh.at[dbuf.at[j]], vb.at[j], sem))
        for h in hs:
            h.wait()
        for j in range(_CR):
            for l in range(8):
                sl = pl.ds(l * 16, 16)
                wbuf[j, sl] = va[j, sl] * vb[j, sl]
        pltpu.sync_copy(wbuf, w_hbm.at[pl.ds(ro, _CR)])
        return carry

    lax.fori_loop(0, EW_ROWS // _CR, chunk1, 0)

    kbase = wid * KW_S
    pltpu.sync_copy(zeros_hbm.at[pl.ds(0, KW_S)], acc)

    def chunk2(ci, carry):
        pltpu.sync_copy(ms2_hbm.at[pl.ds(ci * _CR, _CR)], mbuf)
        pltpu.sync_copy(dv2s_hbm.at[pl.ds(ci * _CR, _CR)], vbuf)
        for j in range(_CR):
            for l in range(8):
                sl = pl.ds(l * 16, 16)
                i16 = mbuf[j, sl]
                v16 = vbuf[j, sl]
                loc = i16 - kbase
                msk = (i16 >= kbase) & (i16 < kbase + KW_S)
                locc = jnp.minimum(jnp.maximum(loc, 0), KW_S - 1)
                plsc.addupdate_scatter(acc, [locc], v16, mask=msk)
        return carry

    lax.fori_loop(0, SROWS // _CR, chunk2, 0)
    pltpu.sync_copy(acc, s2_hbm.at[pl.ds(kbase, KW_S)])


def _sc_edge_weights(src2, dst2, dinv_tbl, m_s2, dv2_s2, zeros_np):
    f = pl.kernel(
        _sc_edge_weights_body,
        out_type=(
            jax.ShapeDtypeStruct((EROWS, 128), jnp.float32),
            jax.ShapeDtypeStruct((NP,), jnp.float32),
        ),
        mesh=_MESH,
        compiler_params=pltpu.CompilerParams(needs_layout_passes=False),
        scratch_types=[
            pltpu.VMEM_SHARED((S_TBL,), jnp.float32),
            pltpu.VMEM((_CR, 128), jnp.int32),
            pltpu.VMEM((_CR, 128), jnp.int32),
            pltpu.VMEM((_CR, 128), jnp.float32),
            pltpu.VMEM((_CR, 128), jnp.float32),
            pltpu.VMEM((_CR, 128), jnp.float32),
            pltpu.VMEM((_CR, 128), jnp.int32),
            pltpu.VMEM((_CR, 128), jnp.float32),
            pltpu.VMEM((KW_S,), jnp.float32),
            pltpu.SemaphoreType.DMA,
        ],
    )
    return f(src2, dst2, dinv_tbl, m_s2, dv2_s2, zeros_np)


def _sc_table_body(nrows, ki_hbm, kp_hbm, val_hbm, zeros_hbm, t_hbm,
                   acc, ib, pb, vb, semS):
    wid = _wid()
    p0 = 2 * wid
    nch = nrows // _CR

    def stage(i, p):
        ro = i * _CR
        pltpu.async_copy(ki_hbm.at[pl.ds(ro, _CR)], ib.at[p], semS)
        pltpu.async_copy(kp_hbm.at[pl.ds(ro, _CR)], pb.at[p], semS)
        pltpu.async_copy(val_hbm.at[pl.ds(ro, _CR)], vb.at[p], semS)

    def drain_stage(i, p):
        ro = i * _CR
        pltpu.make_async_copy(ki_hbm.at[pl.ds(ro, _CR)], ib.at[p], semS).wait()
        pltpu.make_async_copy(kp_hbm.at[pl.ds(ro, _CR)], pb.at[p], semS).wait()
        pltpu.make_async_copy(val_hbm.at[pl.ds(ro, _CR)], vb.at[p], semS).wait()

    stage(0, 0)
    pltpu.sync_copy(zeros_hbm, acc.at[pl.ds(0, NP)])
    pltpu.sync_copy(zeros_hbm, acc.at[pl.ds(NP, NP)])

    def body_one(i, p):
        drain_stage(i, p)

        @pl.when(i + 1 < nch)
        def _():
            stage(i + 1, 1 - p)

        for j in range(_CR):
            for l in range(8):
                sl = pl.ds(l * 16, 16)
                i16 = ib[p, j, sl]
                p16 = pb[p, j, sl]
                v16 = vb[p, j, sl]
                msk = (p16 >> 1) == wid
                idx = i16 + (p16 & 1) * NP
                plsc.addupdate_scatter(acc, [idx], v16, mask=msk)

    def pair(k, carry):
        body_one(2 * k, 0)

        @pl.when(2 * k + 1 < nch)
        def _():
            body_one(2 * k + 1, 1)

        return carry

    lax.fori_loop(0, (nch + 1) // 2, pair, 0)
    pltpu.sync_copy(acc.at[pl.ds(0, NP)], t_hbm.at[pl.ds(p0 * NP, NP)])
    pltpu.sync_copy(acc.at[pl.ds(NP, NP)], t_hbm.at[pl.ds((p0 + 1) * NP, NP)])


def _sc_table(ki2, kp2, val2, nrows, zeros_np):
    f = pl.kernel(
        functools.partial(_sc_table_body, nrows),
        out_type=jax.ShapeDtypeStruct((P * NP,), jnp.float32),
        mesh=_MESH,
        compiler_params=pltpu.CompilerParams(needs_layout_passes=False),
        scratch_types=[
            pltpu.VMEM((2 * NP,), jnp.float32),
            pltpu.VMEM((2, _CR, 128), jnp.int32),
            pltpu.VMEM((2, _CR, 128), jnp.int32),
            pltpu.VMEM((2, _CR, 128), jnp.float32),
            pltpu.SemaphoreType.DMA,
        ],
    )
    return f(ki2, kp2, val2, zeros_np).reshape(P, NP)


_CRK = 16
_NCHK = EROWS // _CRK


def _sc_kbuild_body(md_hbm, ms_hbm, w_hbm, mc_hbm, zeros_hbm, k_hbm,
                    mc_sh, mdb, msb, wb, idx, g, acc0, semS, semG):
    c = lax.axis_index("c")
    s = lax.axis_index("s")

    def stage(i, p):
        ro = i * _CRK
        pltpu.async_copy(md_hbm.at[pl.ds(ro, _CRK)], mdb.at[p], semS)
        pltpu.async_copy(ms_hbm.at[pl.ds(ro, _CRK)], msb.at[p], semS)
        pltpu.async_copy(w_hbm.at[pl.ds(ro, _CRK)], wb.at[p], semS)

    def drain_stage(i, p):
        ro = i * _CRK
        pltpu.make_async_copy(md_hbm.at[pl.ds(ro, _CRK)], mdb.at[p], semS).wait()
        pltpu.make_async_copy(ms_hbm.at[pl.ds(ro, _CRK)], msb.at[p], semS).wait()
        pltpu.make_async_copy(w_hbm.at[pl.ds(ro, _CRK)], wb.at[p], semS).wait()

    for h in range(2):
        @pl.when(s == 0)
        def _load():
            pltpu.sync_copy(
                mc_hbm.at[pl.ds((c * 32 + h * 16) * NP, 16 * NP)], mc_sh)

        plsc.subcore_barrier()

        pltpu.sync_copy(zeros_hbm, acc0)
        off0 = s * NP
        p0 = c * 32 + h * 16 + s

        stage(0, 0)

        def body_one(i, p):
            drain_stage(i, p)
            for j in range(_CRK):
                for l in range(8):
                    sl = pl.ds(l * 16, 16)
                    idx[p, j, sl] = mdb[p, j, sl] + off0
            for j in range(_CRK):
                pltpu.async_copy(mc_sh.at[idx.at[p, j]], g.at[p, j], semG)

            @pl.when(i > 0)
            def _():
                q = 1 - p
                for j in range(_CRK):
                    pltpu.make_async_copy(mc_sh.at[idx.at[q, j]],
                                          g.at[q, j], semG).wait()
                for j in range(_CRK):
                    for l in range(8):
                        sl = pl.ds(l * 16, 16)
                        plsc.addupdate_scatter(
                            acc0, [msb[q, j, sl]], wb[q, j, sl] * g[q, j, sl])

            @pl.when(i + 1 < _NCHK)
            def _():
                stage(i + 1, 1 - p)

        def pair(k, carry):
            body_one(2 * k, 0)
            body_one(2 * k + 1, 1)
            return carry

        lax.fori_loop(0, _NCHK // 2, pair, 0)

        qe = 1
        for j in range(_CRK):
            pltpu.make_async_copy(mc_sh.at[idx.at[qe, j]],
                                  g.at[qe, j], semG).wait()
        for j in range(_CRK):
            for l in range(8):
                sl = pl.ds(l * 16, 16)
                plsc.addupdate_scatter(
                    acc0, [msb[qe, j, sl]], wb[qe, j, sl] * g[qe, j, sl])

        pltpu.sync_copy(acc0, k_hbm.at[pl.ds(p0 * NP, NP)])
        plsc.subcore_barrier()


def _sc_kbuild(md2, ms2, w2, mc_flat, zeros_np):
    f = pl.kernel(
        _sc_kbuild_body,
        out_type=jax.ShapeDtypeStruct((P * NP,), jnp.float32),
        mesh=_MESH,
        compiler_params=pltpu.CompilerParams(needs_layout_passes=False),
        scratch_types=[
            pltpu.VMEM_SHARED((16 * NP,), jnp.float32),
            pltpu.VMEM((2, _CRK, 128), jnp.int32),
            pltpu.VMEM((2, _CRK, 128), jnp.int32),
            pltpu.VMEM((2, _CRK, 128), jnp.float32),
            pltpu.VMEM((2, _CRK, 128), jnp.int32),
            pltpu.VMEM((2, _CRK, 128), jnp.float32),
            pltpu.VMEM((NP,), jnp.float32),
            pltpu.SemaphoreType.DMA,
            pltpu.SemaphoreType.DMA,
        ],
    )
    return f(md2, ms2, w2, mc_flat, zeros_np).reshape(P, NP)


def _tc_reduce_body(g_ref, bmat_ref, ct_ref, x_ref, s2x_ref,
                    w0_ref, b0_ref, wu_ref, bu_ref,
                    mc_ref, sg_ref, p1_ref, mcct_ref, smalls_ref, r_ref):
    i = pl.program_id(0)

    gblk = g_ref[...]
    bblk = bmat_ref[...]
    ctblk = ct_ref[...]
    xblk = x_ref[...]
    s2xblk = s2x_ref[...]

    mblk = gblk + bblk
    cm = jnp.maximum(jnp.sum(ctblk, axis=0, keepdims=True), 1.0)
    mcblk = mblk / cm
    mc_ref[...] = mcblk

    sg = jnp.dot(mblk, xblk, preferred_element_type=jnp.float32)
    p1 = jnp.dot(mcblk, s2xblk, preferred_element_type=jnp.float32)
    mcct = lax.dot_general(mcblk, ctblk, (((1,), (1,)), ((), ())),
                           preferred_element_type=jnp.float32)

    lane = lax.broadcasted_iota(jnp.int32, (P, D), 1)
    rs_m = jnp.sum(mblk, axis=1)
    rs_ct = jnp.sum(ctblk, axis=1)
    smalls = (jnp.where(lane == 0, rs_m[:, None], 0.0)
              + jnp.where(lane == 1, rs_ct[:, None], 0.0))

    @pl.when(i == 0)
    def _init():
        sg_ref[...] = sg
        p1_ref[...] = p1
        mcct_ref[...] = mcct
        smalls_ref[...] = smalls

    @pl.when(i > 0)
    def _acc():
        sg_ref[...] += sg
        p1_ref[...] += p1
        mcct_ref[...] += mcct
        smalls_ref[...] += smalls

    @pl.when(i == GRID - 1)
    def _epilogue():
        sm = smalls_ref[...]
        cnt_b = jnp.sum(jnp.where(lane == 1, sm, 0.0), axis=1)
        cb = jnp.maximum(cnt_b, 1.0)
        sgpool = (jnp.dot(sg_ref[...] / cb[:, None], w0_ref[...],
                          preferred_element_type=jnp.float32)
                  + b0_ref[...])
        r_ref[...] = jax.nn.relu(
            jnp.dot(sgpool, wu_ref[...], preferred_element_type=jnp.float32)
            + bu_ref[...])


def _tc_reduce(G, B, Ct, x, s2x, W0, b0, Wu, bu):
    tbl = pl.BlockSpec((P, BN), lambda i: (0, i))
    xsp = pl.BlockSpec((BN, D), lambda i: (i, 0))
    wsp = pl.BlockSpec((D, D), lambda i: (0, 0))
    bsp = pl.BlockSpec((1, D), lambda i: (0, 0))
    acc = pl.BlockSpec((P, D), lambda i: (0, 0))
    out_shapes = (
        jax.ShapeDtypeStruct((P, NP), jnp.float32),
        jax.ShapeDtypeStruct((P, D), jnp.float32),
        jax.ShapeDtypeStruct((P, D), jnp.float32),
        jax.ShapeDtypeStruct((P, P), jnp.float32),
        jax.ShapeDtypeStruct((P, D), jnp.float32),
        jax.ShapeDtypeStruct((P, D), jnp.float32),
    )
    return pl.pallas_call(
        _tc_reduce_body,
        grid=(GRID,),
        in_specs=[tbl, tbl, tbl, xsp, xsp, wsp, bsp, wsp, bsp],
        out_specs=(tbl, acc, acc, pl.BlockSpec((P, P), lambda i: (0, 0)),
                   acc, acc),
        out_shape=out_shapes,
        compiler_params=pltpu.CompilerParams(
            dimension_semantics=("arbitrary",)),
    )(G, B, Ct, x, s2x, W0, b0, Wu, bu)


def _tc_out_body(k_ref, x_ref, p1_ref, mcct_ref, r_ref, smalls_ref,
                 w0_ref, b0_ref, w1_ref, b1_ref,
                 kx_ref, out_ref):
    i = pl.program_id(0)
    kx = jnp.dot(k_ref[...], x_ref[...], preferred_element_type=jnp.float32)

    @pl.when(i == 0)
    def _init():
        kx_ref[...] = kx

    @pl.when(i > 0)
    def _acc():
        kx_ref[...] += kx

    @pl.when(i == GRID - 1)
    def _epilogue():
        lane = lax.broadcasted_iota(jnp.int32, (P, D), 1)
        sm = smalls_ref[...]
        colsum_m = jnp.sum(jnp.where(lane == 0, sm, 0.0), axis=1)
        cnt_b = jnp.sum(jnp.where(lane == 1, sm, 0.0), axis=1)
        cb = jnp.maximum(cnt_b, 1.0)
        out_sum = (jnp.dot(kx_ref[...] + p1_ref[...], w0_ref[...],
                           preferred_element_type=jnp.float32)
                   + jnp.dot(mcct_ref[...], r_ref[...],
                             preferred_element_type=jnp.float32)
                   + colsum_m[:, None] * b0_ref[...])
        out = (jnp.dot(out_sum / cb[:, None], w1_ref[...],
                       preferred_element_type=jnp.float32)
               + b1_ref[...])
        out_ref[...] = jnp.where(cnt_b[:, None] > 0, out, 0.0)


def _tc_out(K, x, partial1, McCt, r, smalls, W0, b0, W1, b1):
    tbl = pl.BlockSpec((P, BN), lambda i: (0, i))
    xsp = pl.BlockSpec((BN, D), lambda i: (i, 0))
    wsp = pl.BlockSpec((D, D), lambda i: (0, 0))
    bsp = pl.BlockSpec((1, D), lambda i: (0, 0))
    acc = pl.BlockSpec((P, D), lambda i: (0, 0))
    out_shapes = (
        jax.ShapeDtypeStruct((P, D), jnp.float32),
        jax.ShapeDtypeStruct((P, D), jnp.float32),
    )
    res = pl.pallas_call(
        _tc_out_body,
        grid=(GRID,),
        in_specs=[tbl, xsp, acc, pl.BlockSpec((P, P), lambda i: (0, 0)),
                  acc, acc, wsp, bsp, wsp, bsp],
        out_specs=(acc, acc),
        out_shape=out_shapes,
        compiler_params=pltpu.CompilerParams(
            dimension_semantics=("arbitrary",)),
    )(K, x, partial1, McCt, r, smalls, W0, b0, W1, b1)
    return res[1]


def kernel(x, edge_attr, subgraphs_nodes_mapper, subgraphs_edges_mapper,
           combined_subgraphs, subgraphs_batch, W0, b0, W1, b1, Wu, bu):
    m = subgraphs_nodes_mapper
    b = subgraphs_batch
    src = combined_subgraphs[0]
    dst = combined_subgraphs[1]

    zeros_np = jnp.zeros((NP,), jnp.float32)

    src2 = jnp.pad(src, (0, E_PAD - E_SUB),
                   constant_values=N_SUB).reshape(EROWS, 128)
    dst2 = jnp.pad(dst, (0, E_PAD - E_SUB),
                   constant_values=N_SUB).reshape(EROWS, 128)
    m_tbl = jnp.pad(m, (0, S_TBL - N_SUB), constant_values=NP - 1)
    b_tbl = jnp.pad(b, (0, S_TBL - N_SUB), constant_values=-1)

    ms2, md2, bd2, indeg = _sc_gather_maps(src2, dst2, m_tbl, b_tbl, zeros_np)

    deg = 1.0 + indeg[:N_SUB]
    dinv = lax.rsqrt(deg)
    dinv2 = 1.0 / deg
    dinv_tbl = jnp.pad(dinv, (0, S_TBL - N_SUB))

    m_s2 = jnp.pad(m, (0, S_STREAM - N_SUB),
                   constant_values=-1).reshape(SROWS, 128)
    b_s2 = jnp.pad(b, (0, S_STREAM - N_SUB),
                   constant_values=-1).reshape(SROWS, 128)
    dv2_s2 = jnp.pad(dinv2, (0, S_STREAM - N_SUB)).reshape(SROWS, 128)
    ones_s2 = jnp.ones((SROWS, 128), jnp.float32)

    w2, s2 = _sc_edge_weights(src2, dst2, dinv_tbl, m_s2, dv2_s2, zeros_np)

    G = _sc_table(ms2, bd2, w2, EROWS, zeros_np)
    Bm = _sc_table(m_s2, b_s2, dv2_s2, SROWS, zeros_np)
    Ct = _sc_table(m_s2, b_s2, ones_s2, SROWS, zeros_np)

    xp = jnp.pad(x, ((0, NP - N_NODES), (0, 0)))
    s2x = xp * s2[:, None]

    Mc, sg_sum, partial1, McCt, smalls, r = _tc_reduce(
        G, Bm, Ct, xp, s2x, W0, b0[None, :], Wu, bu[None, :])

    K = _sc_kbuild(md2, ms2, w2, Mc.reshape(P * NP), zeros_np)

    return _tc_out(K, xp, partial1, McCt, r, smalls, W0, b0[None, :],
                   W1, b1[None, :])


# speedup vs baseline: 31.3054x; 1.2440x over previous
import functools
import jax
import jax.numpy as jnp
from jax import lax
from jax.experimental import pallas as pl
from jax.experimental.pallas import tpu as pltpu
from jax.experimental.pallas import tpu_sc as plsc

N_NODES = 50000
N_SUB = 150000
E_SUB = 480000
D = 128
P = 64

NP = 51200
BN = 1024
GRID = NP // BN

E_PAD = 491520
EROWS = E_PAD // 128
EW_ROWS = EROWS // 32
S_TBL = 150016
S_STREAM = 150528
SROWS = S_STREAM // 128
S_STREAM2 = 163840
SROWS2 = S_STREAM2 // 128
KW_I = S_TBL // 32
KW_S = NP // 32
_CR = 24

_MESH = plsc.VectorSubcoreMesh(core_axis_name="c", subcore_axis_name="s",
                               num_cores=2, num_subcores=16)


def _wid():
    return lax.axis_index("c") * 16 + lax.axis_index("s")


def _sc_gather_maps_body(src_hbm, dst_hbm, mtbl_hbm, btbl_hbm, zeros_hbm,
                         ms_hbm, md_hbm, bd_hbm, indeg_hbm,
                         m_sh, b_sh, sbuf, dbuf, msv, mdv, bdv, acc, sem):
    s = lax.axis_index("s")
    wid = _wid()

    @pl.when(s == 0)
    def _load():
        pltpu.sync_copy(mtbl_hbm, m_sh)
        pltpu.sync_copy(btbl_hbm, b_sh)

    plsc.subcore_barrier()

    rowbase = wid * EW_ROWS

    def chunk1(ci, carry):
        ro = rowbase + ci * _CR
        pltpu.sync_copy(src_hbm.at[pl.ds(ro, _CR)], sbuf)
        pltpu.sync_copy(dst_hbm.at[pl.ds(ro, _CR)], dbuf)
        hs = []
        for j in range(_CR):
            hs.append(pltpu.async_copy(m_sh.at[sbuf.at[j]], msv.at[j], sem))
            hs.append(pltpu.async_copy(m_sh.at[dbuf.at[j]], mdv.at[j], sem))
            hs.append(pltpu.async_copy(b_sh.at[dbuf.at[j]], bdv.at[j], sem))
        for h in hs:
            h.wait()
        pltpu.sync_copy(msv, ms_hbm.at[pl.ds(ro, _CR)])
        pltpu.sync_copy(mdv, md_hbm.at[pl.ds(ro, _CR)])
        pltpu.sync_copy(bdv, bd_hbm.at[pl.ds(ro, _CR)])
        return carry

    lax.fori_loop(0, EW_ROWS // _CR, chunk1, 0)

    kbase = wid * KW_I
    pltpu.sync_copy(zeros_hbm.at[pl.ds(0, KW_I)], acc)
    ones16 = jnp.ones((16,), jnp.float32)

    def chunk2(ci, carry):
        pltpu.sync_copy(dst_hbm.at[pl.ds(ci * _CR, _CR)], dbuf)
        for j in range(_CR):
            for l in range(8):
                d16 = dbuf[j, pl.ds(l * 16, 16)]
                loc = d16 - kbase
                msk = (d16 >= kbase) & (d16 < kbase + KW_I)
                locc = jnp.minimum(jnp.maximum(loc, 0), KW_I - 1)
                plsc.addupdate_scatter(acc, [locc], ones16, mask=msk)
        return carry

    lax.fori_loop(0, EROWS // _CR, chunk2, 0)
    pltpu.sync_copy(acc, indeg_hbm.at[pl.ds(kbase, KW_I)])


def _sc_gather_maps(src2, dst2, m_tbl, b_tbl, zeros_np):
    f = pl.kernel(
        _sc_gather_maps_body,
        out_type=(
            jax.ShapeDtypeStruct((EROWS, 128), jnp.int32),
            jax.ShapeDtypeStruct((EROWS, 128), jnp.int32),
            jax.ShapeDtypeStruct((EROWS, 128), jnp.int32),
            jax.ShapeDtypeStruct((S_TBL,), jnp.float32),
        ),
        mesh=_MESH,
        compiler_params=pltpu.CompilerParams(needs_layout_passes=False),
        scratch_types=[
            pltpu.VMEM_SHARED((S_TBL,), jnp.int32),
            pltpu.VMEM_SHARED((S_TBL,), jnp.int32),
            pltpu.VMEM((_CR, 128), jnp.int32),
            pltpu.VMEM((_CR, 128), jnp.int32),
            pltpu.VMEM((_CR, 128), jnp.int32),
            pltpu.VMEM((_CR, 128), jnp.int32),
            pltpu.VMEM((_CR, 128), jnp.int32),
            pltpu.VMEM((KW_I,), jnp.float32),
            pltpu.SemaphoreType.DMA,
        ],
    )
    return f(src2, dst2, m_tbl, b_tbl, zeros_np)


def _sc_edge_weights_body(src_hbm, dst_hbm, dinv_hbm, ms2_hbm, dv2s_hbm,
                          zeros_hbm,
                          w_hbm, s2_hbm,
                          dinv_sh, sbuf, dbuf, va, vb, wbuf, mbuf, vbuf,
                          acc, sem):
    s = lax.axis_index("s")
    wid = _wid()

    @pl.when(s == 0)
    def _load():
        pltpu.sync_copy(dinv_hbm, dinv_sh)

    plsc.subcore_barrier()

    rowbase = wid * EW_ROWS

    def chunk1(ci, carry):
        ro = rowbase + ci * _CR
        pltpu.sync_copy(src_hbm.at[pl.ds(ro, _CR)], sbuf)
        pltpu.sync_copy(dst_hbm.at[pl.ds(ro, _CR)], dbuf)
        hs = []
        for j in range(_CR):
            hs.append(pltpu.async_copy(dinv_sh.at[sbuf.at[j]], va.at[j], sem))
            hs.append(pltpu.async_copy(dinv_sh.at[dbuf.at[j]], vb.at[j], sem))
        for h in hs:
            h.wait()
        for j in range(_CR):
            for l in range(8):
                sl = pl.ds(l * 16, 16)
                wbuf[j, sl] = va[j, sl] * vb[j, sl]
        pltpu.sync_copy(wbuf, w_hbm.at[pl.ds(ro, _CR)])
        return carry

    lax.fori_loop(0, EW_ROWS // _CR, chunk1, 0)

    kbase = wid * KW_S
    pltpu.sync_copy(zeros_hbm.at[pl.ds(0, KW_S)], acc)

    def chunk2(ci, carry):
        pltpu.sync_copy(ms2_hbm.at[pl.ds(ci * _CR, _CR)], mbuf)
        pltpu.sync_copy(dv2s_hbm.at[pl.ds(ci * _CR, _CR)], vbuf)
        for j in range(_CR):
            for l in range(8):
                sl = pl.ds(l * 16, 16)
                i16 = mbuf[j, sl]
                v16 = vbuf[j, sl]
                loc = i16 - kbase
                msk = (i16 >= kbase) & (i16 < kbase + KW_S)
                locc = jnp.minimum(jnp.maximum(loc, 0), KW_S - 1)
                plsc.addupdate_scatter(acc, [locc], v16, mask=msk)
        return carry

    lax.fori_loop(0, SROWS // _CR, chunk2, 0)
    pltpu.sync_copy(acc, s2_hbm.at[pl.ds(kbase, KW_S)])


def _sc_edge_weights(src2, dst2, dinv_tbl, m_s2, dv2_s2, zeros_np):
    f = pl.kernel(
        _sc_edge_weights_body,
        out_type=(
            jax.ShapeDtypeStruct((EROWS, 128), jnp.float32),
            jax.ShapeDtypeStruct((NP,), jnp.float32),
        ),
        mesh=_MESH,
        compiler_params=pltpu.CompilerParams(needs_layout_passes=False),
        scratch_types=[
            pltpu.VMEM_SHARED((S_TBL,), jnp.float32),
            pltpu.VMEM((_CR, 128), jnp.int32),
            pltpu.VMEM((_CR, 128), jnp.int32),
            pltpu.VMEM((_CR, 128), jnp.float32),
            pltpu.VMEM((_CR, 128), jnp.float32),
            pltpu.VMEM((_CR, 128), jnp.float32),
            pltpu.VMEM((_CR, 128), jnp.int32),
            pltpu.VMEM((_CR, 128), jnp.float32),
            pltpu.VMEM((KW_S,), jnp.float32),
            pltpu.SemaphoreType.DMA,
        ],
    )
    return f(src2, dst2, dinv_tbl, m_s2, dv2_s2, zeros_np)


def _sc_table_body(nrows, ki_hbm, kp_hbm, val_hbm, zeros_hbm, t_hbm,
                   acc, ib, pb, vb, semS):
    wid = _wid()
    p0 = 2 * wid
    nch = nrows // _CR

    def stage(i, p):
        ro = i * _CR
        pltpu.async_copy(ki_hbm.at[pl.ds(ro, _CR)], ib.at[p], semS)
        pltpu.async_copy(kp_hbm.at[pl.ds(ro, _CR)], pb.at[p], semS)
        pltpu.async_copy(val_hbm.at[pl.ds(ro, _CR)], vb.at[p], semS)

    def drain_stage(i, p):
        ro = i * _CR
        pltpu.make_async_copy(ki_hbm.at[pl.ds(ro, _CR)], ib.at[p], semS).wait()
        pltpu.make_async_copy(kp_hbm.at[pl.ds(ro, _CR)], pb.at[p], semS).wait()
        pltpu.make_async_copy(val_hbm.at[pl.ds(ro, _CR)], vb.at[p], semS).wait()

    stage(0, 0)
    pltpu.sync_copy(zeros_hbm, acc.at[pl.ds(0, NP)])
    pltpu.sync_copy(zeros_hbm, acc.at[pl.ds(NP, NP)])

    def body_one(i, p):
        drain_stage(i, p)

        @pl.when(i + 1 < nch)
        def _():
            stage(i + 1, 1 - p)

        for j in range(_CR):
            for l in range(8):
                sl = pl.ds(l * 16, 16)
                i16 = ib[p, j, sl]
                p16 = pb[p, j, sl]
                v16 = vb[p, j, sl]
                msk = (p16 >> 1) == wid
                idx = i16 + (p16 & 1) * NP
                plsc.addupdate_scatter(acc, [idx], v16, mask=msk)

    def pair(k, carry):
        body_one(2 * k, 0)

        @pl.when(2 * k + 1 < nch)
        def _():
            body_one(2 * k + 1, 1)

        return carry

    lax.fori_loop(0, (nch + 1) // 2, pair, 0)
    pltpu.sync_copy(acc.at[pl.ds(0, NP)], t_hbm.at[pl.ds(p0 * NP, NP)])
    pltpu.sync_copy(acc.at[pl.ds(NP, NP)], t_hbm.at[pl.ds((p0 + 1) * NP, NP)])


def _sc_table(ki2, kp2, val2, nrows, zeros_np):
    f = pl.kernel(
        functools.partial(_sc_table_body, nrows),
        out_type=jax.ShapeDtypeStruct((P * NP,), jnp.float32),
        mesh=_MESH,
        compiler_params=pltpu.CompilerParams(needs_layout_passes=False),
        scratch_types=[
            pltpu.VMEM((2 * NP,), jnp.float32),
            pltpu.VMEM((2, _CR, 128), jnp.int32),
            pltpu.VMEM((2, _CR, 128), jnp.int32),
            pltpu.VMEM((2, _CR, 128), jnp.float32),
            pltpu.SemaphoreType.DMA,
        ],
    )
    return f(ki2, kp2, val2, zeros_np).reshape(P, NP)


NPH = NP // 2
_TSZ = P * NPH
_TLSZ = _TSZ // 16


def _sc_table2_body(rows_per_tile, crows, ki_hbm, kp_hbm, val_hbm, zeros_hbm,
                    t_hbm, tbl_sh, ib, pb, vb, idxb, semS):
    c = lax.axis_index("c")
    s = lax.axis_index("s")
    nch = rows_per_tile // crows
    rowbase = s * rows_per_tile
    nbase = c * NPH

    def stage(i, p):
        ro = rowbase + i * crows
        pltpu.async_copy(ki_hbm.at[pl.ds(ro, crows)], ib.at[p], semS)
        pltpu.async_copy(kp_hbm.at[pl.ds(ro, crows)], pb.at[p], semS)
        pltpu.async_copy(val_hbm.at[pl.ds(ro, crows)], vb.at[p], semS)

    def drain_stage(i, p):
        ro = rowbase + i * crows
        pltpu.make_async_copy(ki_hbm.at[pl.ds(ro, crows)], ib.at[p], semS).wait()
        pltpu.make_async_copy(kp_hbm.at[pl.ds(ro, crows)], pb.at[p], semS).wait()
        pltpu.make_async_copy(val_hbm.at[pl.ds(ro, crows)], vb.at[p], semS).wait()

    stage(0, 0)
    pltpu.sync_copy(zeros_hbm, tbl_sh.at[pl.ds(s * _TLSZ, NP)])
    pltpu.sync_copy(zeros_hbm, tbl_sh.at[pl.ds(s * _TLSZ + NP, NP)])

    @pl.when(s == 0)
    def _zt():
        pltpu.sync_copy(zeros_hbm.at[pl.ds(0, 128)], tbl_sh.at[pl.ds(_TSZ, 128)])

    plsc.subcore_barrier()

    lanes = lax.broadcasted_iota(jnp.int32, (16,), 0)

    def body_one(i, p):
        drain_stage(i, p)

        @pl.when(i + 1 < nch)
        def _():
            stage(i + 1, 1 - p)

        for j in range(crows):
            for l in range(8):
                sl = pl.ds(l * 16, 16)
                i16 = ib[p, j, sl]
                p16 = pb[p, j, sl]
                loc = i16 - nbase
                valid = (loc >= 0) & (loc < NPH) & (p16 >= 0)
                idxb[j, sl] = jnp.where(valid, p16 * NPH + loc, _TSZ + lanes)
        for j in range(crows):
            pltpu.sync_copy(vb.at[p, j], tbl_sh.at[idxb.at[j]], add=True)

    def pair(k, carry):
        body_one(2 * k, 0)
        body_one(2 * k + 1, 1)
        return carry

    lax.fori_loop(0, nch // 2, pair, 0)
    plsc.subcore_barrier()
    pltpu.sync_copy(tbl_sh.at[pl.ds(s * _TLSZ, _TLSZ)],
                    t_hbm.at[pl.ds(c * _TSZ + s * _TLSZ, _TLSZ)])


def _sc_table2(ki2, kp2, val2, rows_per_tile, crows, zeros_np):
    f = pl.kernel(
        functools.partial(_sc_table2_body, rows_per_tile, crows),
        out_type=jax.ShapeDtypeStruct((2 * _TSZ,), jnp.float32),
        mesh=_MESH,
        compiler_params=pltpu.CompilerParams(needs_layout_passes=False),
        scratch_types=[
            pltpu.VMEM_SHARED((_TSZ + 128,), jnp.float32),
            pltpu.VMEM((2, crows, 128), jnp.int32),
            pltpu.VMEM((2, crows, 128), jnp.int32),
            pltpu.VMEM((2, crows, 128), jnp.float32),
            pltpu.VMEM((crows, 128), jnp.int32),
            pltpu.SemaphoreType.DMA,
        ],
    )
    t = f(ki2, kp2, val2, zeros_np).reshape(2, P, NPH)
    return jnp.concatenate([t[0], t[1]], axis=1)


_CRK = 16
_NCHK = EROWS // _CRK


def _sc_kbuild_body(md_hbm, ms_hbm, w_hbm, mc_hbm, zeros_hbm, k_hbm,
                    mc_sh, mdb, msb, wb, idx, g, acc0, semS, semG):
    c = lax.axis_index("c")
    s = lax.axis_index("s")

    def stage(i, p):
        ro = i * _CRK
        pltpu.async_copy(md_hbm.at[pl.ds(ro, _CRK)], mdb.at[p], semS)
        pltpu.async_copy(ms_hbm.at[pl.ds(ro, _CRK)], msb.at[p], semS)
        pltpu.async_copy(w_hbm.at[pl.ds(ro, _CRK)], wb.at[p], semS)

    def drain_stage(i, p):
        ro = i * _CRK
        pltpu.make_async_copy(md_hbm.at[pl.ds(ro, _CRK)], mdb.at[p], semS).wait()
        pltpu.make_async_copy(ms_hbm.at[pl.ds(ro, _CRK)], msb.at[p], semS).wait()
        pltpu.make_async_copy(w_hbm.at[pl.ds(ro, _CRK)], wb.at[p], semS).wait()

    for h in range(2):
        @pl.when(s == 0)
        def _load():
            pltpu.sync_copy(
                mc_hbm.at[pl.ds((c * 32 + h * 16) * NP, 16 * NP)], mc_sh)

        plsc.subcore_barrier()

        pltpu.sync_copy(zeros_hbm, acc0)
        off0 = s * NP
        p0 = c * 32 + h * 16 + s

        stage(0, 0)

        def body_one(i, p):
            drain_stage(i, p)
            for j in range(_CRK):
                for l in range(8):
                    sl = pl.ds(l * 16, 16)
                    idx[p, j, sl] = mdb[p, j, sl] + off0
            for j in range(_CRK):
                pltpu.async_copy(mc_sh.at[idx.at[p, j]], g.at[p, j], semG)

            @pl.when(i > 0)
            def _():
                q = 1 - p
                for j in range(_CRK):
                    pltpu.make_async_copy(mc_sh.at[idx.at[q, j]],
                                          g.at[q, j], semG).wait()
                for j in range(_CRK):
                    for l in range(8):
                        sl = pl.ds(l * 16, 16)
                        plsc.addupdate_scatter(
                            acc0, [msb[q, j, sl]], wb[q, j, sl] * g[q, j, sl])

            @pl.when(i + 1 < _NCHK)
            def _():
                stage(i + 1, 1 - p)

        def pair(k, carry):
            body_one(2 * k, 0)
            body_one(2 * k + 1, 1)
            return carry

        lax.fori_loop(0, _NCHK // 2, pair, 0)

        qe = 1
        for j in range(_CRK):
            pltpu.make_async_copy(mc_sh.at[idx.at[qe, j]],
                                  g.at[qe, j], semG).wait()
        for j in range(_CRK):
            for l in range(8):
                sl = pl.ds(l * 16, 16)
                plsc.addupdate_scatter(
                    acc0, [msb[qe, j, sl]], wb[qe, j, sl] * g[qe, j, sl])

        pltpu.sync_copy(acc0, k_hbm.at[pl.ds(p0 * NP, NP)])
        plsc.subcore_barrier()


def _sc_kbuild(md2, ms2, w2, mc_flat, zeros_np):
    f = pl.kernel(
        _sc_kbuild_body,
        out_type=jax.ShapeDtypeStruct((P * NP,), jnp.float32),
        mesh=_MESH,
        compiler_params=pltpu.CompilerParams(needs_layout_passes=False),
        scratch_types=[
            pltpu.VMEM_SHARED((16 * NP,), jnp.float32),
            pltpu.VMEM((2, _CRK, 128), jnp.int32),
            pltpu.VMEM((2, _CRK, 128), jnp.int32),
            pltpu.VMEM((2, _CRK, 128), jnp.float32),
            pltpu.VMEM((2, _CRK, 128), jnp.int32),
            pltpu.VMEM((2, _CRK, 128), jnp.float32),
            pltpu.VMEM((NP,), jnp.float32),
            pltpu.SemaphoreType.DMA,
            pltpu.SemaphoreType.DMA,
        ],
    )
    return f(md2, ms2, w2, mc_flat, zeros_np).reshape(P, NP)


def _tc_reduce_body(g_ref, bmat_ref, ct_ref, x_ref, s2x_ref,
                    w0_ref, b0_ref, wu_ref, bu_ref,
                    mc_ref, sg_ref, p1_ref, mcct_ref, smalls_ref, r_ref):
    i = pl.program_id(0)

    gblk = g_ref[...]
    bblk = bmat_ref[...]
    ctblk = ct_ref[...]
    xblk = x_ref[...]
    s2xblk = s2x_ref[...]

    mblk = gblk + bblk
    cm = jnp.maximum(jnp.sum(ctblk, axis=0, keepdims=True), 1.0)
    mcblk = mblk / cm
    mc_ref[...] = mcblk

    sg = jnp.dot(mblk, xblk, preferred_element_type=jnp.float32)
    p1 = jnp.dot(mcblk, s2xblk, preferred_element_type=jnp.float32)
    mcct = lax.dot_general(mcblk, ctblk, (((1,), (1,)), ((), ())),
                           preferred_element_type=jnp.float32)

    lane = lax.broadcasted_iota(jnp.int32, (P, D), 1)
    rs_m = jnp.sum(mblk, axis=1)
    rs_ct = jnp.sum(ctblk, axis=1)
    smalls = (jnp.where(lane == 0, rs_m[:, None], 0.0)
              + jnp.where(lane == 1, rs_ct[:, None], 0.0))

    @pl.when(i == 0)
    def _init():
        sg_ref[...] = sg
        p1_ref[...] = p1
        mcct_ref[...] = mcct
        smalls_ref[...] = smalls

    @pl.when(i > 0)
    def _acc():
        sg_ref[...] += sg
        p1_ref[...] += p1
        mcct_ref[...] += mcct
        smalls_ref[...] += smalls

    @pl.when(i == GRID - 1)
    def _epilogue():
        sm = smalls_ref[...]
        cnt_b = jnp.sum(jnp.where(lane == 1, sm, 0.0), axis=1)
        cb = jnp.maximum(cnt_b, 1.0)
        sgpool = (jnp.dot(sg_ref[...] / cb[:, None], w0_ref[...],
                          preferred_element_type=jnp.float32)
                  + b0_ref[...])
        r_ref[...] = jax.nn.relu(
            jnp.dot(sgpool, wu_ref[...], preferred_element_type=jnp.float32)
            + bu_ref[...])


def _tc_reduce(G, B, Ct, x, s2x, W0, b0, Wu, bu):
    tbl = pl.BlockSpec((P, BN), lambda i: (0, i))
    xsp = pl.BlockSpec((BN, D), lambda i: (i, 0))
    wsp = pl.BlockSpec((D, D), lambda i: (0, 0))
    bsp = pl.BlockSpec((1, D), lambda i: (0, 0))
    acc = pl.BlockSpec((P, D), lambda i: (0, 0))
    out_shapes = (
        jax.ShapeDtypeStruct((P, NP), jnp.float32),
        jax.ShapeDtypeStruct((P, D), jnp.float32),
        jax.ShapeDtypeStruct((P, D), jnp.float32),
        jax.ShapeDtypeStruct((P, P), jnp.float32),
        jax.ShapeDtypeStruct((P, D), jnp.float32),
        jax.ShapeDtypeStruct((P, D), jnp.float32),
    )
    return pl.pallas_call(
        _tc_reduce_body,
        grid=(GRID,),
        in_specs=[tbl, tbl, tbl, xsp, xsp, wsp, bsp, wsp, bsp],
        out_specs=(tbl, acc, acc, pl.BlockSpec((P, P), lambda i: (0, 0)),
                   acc, acc),
        out_shape=out_shapes,
        compiler_params=pltpu.CompilerParams(
            dimension_semantics=("arbitrary",)),
    )(G, B, Ct, x, s2x, W0, b0, Wu, bu)


def _tc_out_body(k_ref, x_ref, p1_ref, mcct_ref, r_ref, smalls_ref,
                 w0_ref, b0_ref, w1_ref, b1_ref,
                 kx_ref, out_ref):
    i = pl.program_id(0)
    kx = jnp.dot(k_ref[...], x_ref[...], preferred_element_type=jnp.float32)

    @pl.when(i == 0)
    def _init():
        kx_ref[...] = kx

    @pl.when(i > 0)
    def _acc():
        kx_ref[...] += kx

    @pl.when(i == GRID - 1)
    def _epilogue():
        lane = lax.broadcasted_iota(jnp.int32, (P, D), 1)
        sm = smalls_ref[...]
        colsum_m = jnp.sum(jnp.where(lane == 0, sm, 0.0), axis=1)
        cnt_b = jnp.sum(jnp.where(lane == 1, sm, 0.0), axis=1)
        cb = jnp.maximum(cnt_b, 1.0)
        out_sum = (jnp.dot(kx_ref[...] + p1_ref[...], w0_ref[...],
                           preferred_element_type=jnp.float32)
                   + jnp.dot(mcct_ref[...], r_ref[...],
                             preferred_element_type=jnp.float32)
                   + colsum_m[:, None] * b0_ref[...])
        out = (jnp.dot(out_sum / cb[:, None], w1_ref[...],
                       preferred_element_type=jnp.float32)
               + b1_ref[...])
        out_ref[...] = jnp.where(cnt_b[:, None] > 0, out, 0.0)


def _tc_out(K, x, partial1, McCt, r, smalls, W0, b0, W1, b1):
    tbl = pl.BlockSpec((P, BN), lambda i: (0, i))
    xsp = pl.BlockSpec((BN, D), lambda i: (i, 0))
    wsp = pl.BlockSpec((D, D), lambda i: (0, 0))
    bsp = pl.BlockSpec((1, D), lambda i: (0, 0))
    acc = pl.BlockSpec((P, D), lambda i: (0, 0))
    out_shapes = (
        jax.ShapeDtypeStruct((P, D), jnp.float32),
        jax.ShapeDtypeStruct((P, D), jnp.float32),
    )
    res = pl.pallas_call(
        _tc_out_body,
        grid=(GRID,),
        in_specs=[tbl, xsp, acc, pl.BlockSpec((P, P), lambda i: (0, 0)),
                  acc, acc, wsp, bsp, wsp, bsp],
        out_specs=(acc, acc),
        out_shape=out_shapes,
        compiler_params=pltpu.CompilerParams(
            dimension_semantics=("arbitrary",)),
    )(K, x, partial1, McCt, r, smalls, W0, b0, W1, b1)
    return res[1]


def kernel(x, edge_attr, subgraphs_nodes_mapper, subgraphs_edges_mapper,
           combined_subgraphs, subgraphs_batch, W0, b0, W1, b1, Wu, bu):
    m = subgraphs_nodes_mapper
    b = subgraphs_batch
    src = combined_subgraphs[0]
    dst = combined_subgraphs[1]

    zeros_np = jnp.zeros((NP,), jnp.float32)

    src2 = jnp.pad(src, (0, E_PAD - E_SUB),
                   constant_values=N_SUB).reshape(EROWS, 128)
    dst2 = jnp.pad(dst, (0, E_PAD - E_SUB),
                   constant_values=N_SUB).reshape(EROWS, 128)
    m_tbl = jnp.pad(m, (0, S_TBL - N_SUB), constant_values=NP - 1)
    b_tbl = jnp.pad(b, (0, S_TBL - N_SUB), constant_values=-1)

    ms2, md2, bd2, indeg = _sc_gather_maps(src2, dst2, m_tbl, b_tbl, zeros_np)

    deg = 1.0 + indeg[:N_SUB]
    dinv = lax.rsqrt(deg)
    dinv2 = 1.0 / deg
    dinv_tbl = jnp.pad(dinv, (0, S_TBL - N_SUB))

    m_s2 = jnp.pad(m, (0, S_STREAM - N_SUB),
                   constant_values=-1).reshape(SROWS, 128)
    dv2_s2 = jnp.pad(dinv2, (0, S_STREAM - N_SUB)).reshape(SROWS, 128)

    w2, s2 = _sc_edge_weights(src2, dst2, dinv_tbl, m_s2, dv2_s2, zeros_np)

    m_t2 = jnp.pad(m, (0, S_STREAM2 - N_SUB),
                   constant_values=-1).reshape(SROWS2, 128)
    b_t2 = jnp.pad(b, (0, S_STREAM2 - N_SUB),
                   constant_values=-1).reshape(SROWS2, 128)
    dv2_t2 = jnp.pad(dinv2, (0, S_STREAM2 - N_SUB)).reshape(SROWS2, 128)
    ones_t2 = jnp.ones((SROWS2, 128), jnp.float32)

    G = _sc_table2(ms2, bd2, w2, EROWS // 16, 24, zeros_np)
    Bm = _sc_table2(m_t2, b_t2, dv2_t2, SROWS2 // 16, 8, zeros_np)
    Ct = _sc_table2(m_t2, b_t2, ones_t2, SROWS2 // 16, 8, zeros_np)

    xp = jnp.pad(x, ((0, NP - N_NODES), (0, 0)))
    s2x = xp * s2[:, None]

    Mc, sg_sum, partial1, McCt, smalls, r = _tc_reduce(
        G, Bm, Ct, xp, s2x, W0, b0[None, :], Wu, bu[None, :])

    K = _sc_kbuild(md2, ms2, w2, Mc.reshape(P * NP), zeros_np)

    return _tc_out(K, xp, partial1, McCt, r, smalls, W0, b0[None, :],
                   W1, b1[None, :])


# speedup vs baseline: 40.1044x; 1.2811x over previous
import functools
import jax
import jax.numpy as jnp
from jax import lax
from jax.experimental import pallas as pl
from jax.experimental.pallas import tpu as pltpu
from jax.experimental.pallas import tpu_sc as plsc

N_NODES = 50000
N_SUB = 150000
E_SUB = 480000
D = 128
P = 64

NP = 51200
BN = 1024
GRID = NP // BN

E_PAD = 491520
EROWS = E_PAD // 128
EW_ROWS = EROWS // 32
S_TBL = 150016
S_STREAM = 150528
SROWS = S_STREAM // 128
S_STREAM2 = 163840
SROWS2 = S_STREAM2 // 128
KW_I = S_TBL // 32
KW_S = NP // 32
_CR = 24

_MESH = plsc.VectorSubcoreMesh(core_axis_name="c", subcore_axis_name="s",
                               num_cores=2, num_subcores=16)


def _wid():
    return lax.axis_index("c") * 16 + lax.axis_index("s")


def _sc_gather_maps_body(src_hbm, dst_hbm, mtbl_hbm, btbl_hbm, zeros2_hbm,
                         ms_hbm, md_hbm, bd_hbm, indeg_hbm,
                         m_sh, b_sh, ideg_sh, sbuf, dbuf, msv, mdv, bdv,
                         onesb, sem):
    s = lax.axis_index("s")
    wid = _wid()

    @pl.when(s == 0)
    def _load():
        pltpu.sync_copy(mtbl_hbm, m_sh)
        pltpu.sync_copy(btbl_hbm, b_sh)

    plsc.subcore_barrier()

    rowbase = wid * EW_ROWS

    def chunk1(ci, carry):
        ro = rowbase + ci * _CR
        pltpu.sync_copy(src_hbm.at[pl.ds(ro, _CR)], sbuf)
        pltpu.sync_copy(dst_hbm.at[pl.ds(ro, _CR)], dbuf)
        hs = []
        for j in range(_CR):
            hs.append(pltpu.async_copy(m_sh.at[sbuf.at[j]], msv.at[j], sem))
            hs.append(pltpu.async_copy(m_sh.at[dbuf.at[j]], mdv.at[j], sem))
            hs.append(pltpu.async_copy(b_sh.at[dbuf.at[j]], bdv.at[j], sem))
        for h in hs:
            h.wait()
        pltpu.sync_copy(msv, ms_hbm.at[pl.ds(ro, _CR)])
        pltpu.sync_copy(mdv, md_hbm.at[pl.ds(ro, _CR)])
        pltpu.sync_copy(bdv, bd_hbm.at[pl.ds(ro, _CR)])
        return carry

    lax.fori_loop(0, EW_ROWS // _CR, chunk1, 0)

    c = lax.axis_index("c")
    for l in range(8):
        onesb[0, pl.ds(l * 16, 16)] = jnp.ones((16,), jnp.float32)
    zch = S_TBL // 4
    @pl.when(s < 4)
    def _zi():
        pltpu.sync_copy(zeros2_hbm.at[pl.ds(0, zch)],
                        ideg_sh.at[pl.ds(s * zch, zch)])
    plsc.subcore_barrier()

    rb2 = c * (EROWS // 2) + s * (EROWS // 32)

    def chunk2(ci, carry):
        pltpu.sync_copy(dst_hbm.at[pl.ds(rb2 + ci * _CR, _CR)], dbuf)
        for j in range(_CR):
            pltpu.sync_copy(onesb.at[0], ideg_sh.at[dbuf.at[j]], add=True)
        return carry

    lax.fori_loop(0, EROWS // 32 // _CR, chunk2, 0)
    plsc.subcore_barrier()

    @pl.when(s < 4)
    def _wb():
        pltpu.sync_copy(ideg_sh.at[pl.ds(s * zch, zch)],
                        indeg_hbm.at[pl.ds(c * S_TBL + s * zch, zch)])


def _sc_gather_maps(src2, dst2, m_tbl, b_tbl, zeros_np):
    f = pl.kernel(
        _sc_gather_maps_body,
        out_type=(
            jax.ShapeDtypeStruct((EROWS, 128), jnp.int32),
            jax.ShapeDtypeStruct((EROWS, 128), jnp.int32),
            jax.ShapeDtypeStruct((EROWS, 128), jnp.int32),
            jax.ShapeDtypeStruct((2 * S_TBL,), jnp.float32),
        ),
        mesh=_MESH,
        compiler_params=pltpu.CompilerParams(needs_layout_passes=False),
        scratch_types=[
            pltpu.VMEM_SHARED((S_TBL,), jnp.int32),
            pltpu.VMEM_SHARED((S_TBL,), jnp.int32),
            pltpu.VMEM_SHARED((S_TBL,), jnp.float32),
            pltpu.VMEM((_CR, 128), jnp.int32),
            pltpu.VMEM((_CR, 128), jnp.int32),
            pltpu.VMEM((_CR, 128), jnp.int32),
            pltpu.VMEM((_CR, 128), jnp.int32),
            pltpu.VMEM((_CR, 128), jnp.int32),
            pltpu.VMEM((1, 128), jnp.float32),
            pltpu.SemaphoreType.DMA,
        ],
    )
    return f(src2, dst2, m_tbl, b_tbl, zeros_np)


def _sc_edge_weights_body(src_hbm, dst_hbm, dinv_hbm, ms2_hbm, dv2s_hbm,
                          zeros_hbm,
                          w_hbm, s2_hbm,
                          dinv_sh, s2_sh, sbuf, dbuf, va, vb, wbuf, mbuf,
                          vbuf, idxb2, sem):
    s = lax.axis_index("s")
    wid = _wid()

    @pl.when(s == 0)
    def _load():
        pltpu.sync_copy(dinv_hbm, dinv_sh)

    plsc.subcore_barrier()

    rowbase = wid * EW_ROWS

    def chunk1(ci, carry):
        ro = rowbase + ci * _CR
        pltpu.sync_copy(src_hbm.at[pl.ds(ro, _CR)], sbuf)
        pltpu.sync_copy(dst_hbm.at[pl.ds(ro, _CR)], dbuf)
        hs = []
        for j in range(_CR):
            hs.append(pltpu.async_copy(dinv_sh.at[sbuf.at[j]], va.at[j], sem))
            hs.append(pltpu.async_copy(dinv_sh.at[dbuf.at[j]], vb.at[j], sem))
        for h in hs:
            h.wait()
        for j in range(_CR):
            for l in range(8):
                sl = pl.ds(l * 16, 16)
                wbuf[j, sl] = va[j, sl] * vb[j, sl]
        pltpu.sync_copy(wbuf, w_hbm.at[pl.ds(ro, _CR)])
        return carry

    lax.fori_loop(0, EW_ROWS // _CR, chunk1, 0)

    c = lax.axis_index("c")
    lanes = lax.broadcasted_iota(jnp.int32, (16,), 0)
    psz = NP // 16
    pltpu.sync_copy(zeros_hbm.at[pl.ds(0, psz)],
                    s2_sh.at[pl.ds(s * psz, psz)])

    @pl.when(s == 0)
    def _zt():
        pltpu.sync_copy(zeros_hbm.at[pl.ds(0, 128)], s2_sh.at[pl.ds(NP, 128)])

    plsc.subcore_barrier()

    rb2 = c * (SROWS2 // 2) + s * (SROWS2 // 32)

    def chunk2(ci, carry):
        ro = rb2 + ci * 8
        pltpu.sync_copy(ms2_hbm.at[pl.ds(ro, 8)], mbuf)
        pltpu.sync_copy(dv2s_hbm.at[pl.ds(ro, 8)], vbuf)
        for j in range(8):
            for l in range(8):
                sl = pl.ds(l * 16, 16)
                i16 = mbuf[j, sl]
                idxb2[j, sl] = jnp.where(i16 >= 0, i16, NP + lanes)
        for j in range(8):
            pltpu.sync_copy(vbuf.at[j], s2_sh.at[idxb2.at[j]], add=True)
        return carry

    lax.fori_loop(0, SROWS2 // 32 // 8, chunk2, 0)
    plsc.subcore_barrier()
    pltpu.sync_copy(s2_sh.at[pl.ds(s * psz, psz)],
                    s2_hbm.at[pl.ds(c * NP + s * psz, psz)])


def _sc_edge_weights(src2, dst2, dinv_tbl, m_s2, dv2_s2, zeros_np):
    f = pl.kernel(
        _sc_edge_weights_body,
        out_type=(
            jax.ShapeDtypeStruct((EROWS, 128), jnp.float32),
            jax.ShapeDtypeStruct((2 * NP,), jnp.float32),
        ),
        mesh=_MESH,
        compiler_params=pltpu.CompilerParams(needs_layout_passes=False),
        scratch_types=[
            pltpu.VMEM_SHARED((S_TBL,), jnp.float32),
            pltpu.VMEM_SHARED((NP + 128,), jnp.float32),
            pltpu.VMEM((_CR, 128), jnp.int32),
            pltpu.VMEM((_CR, 128), jnp.int32),
            pltpu.VMEM((_CR, 128), jnp.float32),
            pltpu.VMEM((_CR, 128), jnp.float32),
            pltpu.VMEM((_CR, 128), jnp.float32),
            pltpu.VMEM((8, 128), jnp.int32),
            pltpu.VMEM((8, 128), jnp.float32),
            pltpu.VMEM((8, 128), jnp.int32),
            pltpu.SemaphoreType.DMA,
        ],
    )
    return f(src2, dst2, dinv_tbl, m_s2, dv2_s2, zeros_np)


def _sc_table_body(nrows, ki_hbm, kp_hbm, val_hbm, zeros_hbm, t_hbm,
                   acc, ib, pb, vb, semS):
    wid = _wid()
    p0 = 2 * wid
    nch = nrows // _CR

    def stage(i, p):
        ro = i * _CR
        pltpu.async_copy(ki_hbm.at[pl.ds(ro, _CR)], ib.at[p], semS)
        pltpu.async_copy(kp_hbm.at[pl.ds(ro, _CR)], pb.at[p], semS)
        pltpu.async_copy(val_hbm.at[pl.ds(ro, _CR)], vb.at[p], semS)

    def drain_stage(i, p):
        ro = i * _CR
        pltpu.make_async_copy(ki_hbm.at[pl.ds(ro, _CR)], ib.at[p], semS).wait()
        pltpu.make_async_copy(kp_hbm.at[pl.ds(ro, _CR)], pb.at[p], semS).wait()
        pltpu.make_async_copy(val_hbm.at[pl.ds(ro, _CR)], vb.at[p], semS).wait()

    stage(0, 0)
    pltpu.sync_copy(zeros_hbm, acc.at[pl.ds(0, NP)])
    pltpu.sync_copy(zeros_hbm, acc.at[pl.ds(NP, NP)])

    def body_one(i, p):
        drain_stage(i, p)

        @pl.when(i + 1 < nch)
        def _():
            stage(i + 1, 1 - p)

        for j in range(_CR):
            for l in range(8):
                sl = pl.ds(l * 16, 16)
                i16 = ib[p, j, sl]
                p16 = pb[p, j, sl]
                v16 = vb[p, j, sl]
                msk = (p16 >> 1) == wid
                idx = i16 + (p16 & 1) * NP
                plsc.addupdate_scatter(acc, [idx], v16, mask=msk)

    def pair(k, carry):
        body_one(2 * k, 0)

        @pl.when(2 * k + 1 < nch)
        def _():
            body_one(2 * k + 1, 1)

        return carry

    lax.fori_loop(0, (nch + 1) // 2, pair, 0)
    pltpu.sync_copy(acc.at[pl.ds(0, NP)], t_hbm.at[pl.ds(p0 * NP, NP)])
    pltpu.sync_copy(acc.at[pl.ds(NP, NP)], t_hbm.at[pl.ds((p0 + 1) * NP, NP)])


def _sc_table(ki2, kp2, val2, nrows, zeros_np):
    f = pl.kernel(
        functools.partial(_sc_table_body, nrows),
        out_type=jax.ShapeDtypeStruct((P * NP,), jnp.float32),
        mesh=_MESH,
        compiler_params=pltpu.CompilerParams(needs_layout_passes=False),
        scratch_types=[
            pltpu.VMEM((2 * NP,), jnp.float32),
            pltpu.VMEM((2, _CR, 128), jnp.int32),
            pltpu.VMEM((2, _CR, 128), jnp.int32),
            pltpu.VMEM((2, _CR, 128), jnp.float32),
            pltpu.SemaphoreType.DMA,
        ],
    )
    return f(ki2, kp2, val2, zeros_np).reshape(P, NP)


NPH = NP // 2
_TSZ = P * NPH
_TLSZ = _TSZ // 16


def _sc_table2_body(rows_per_tile, crows, ki_hbm, kp_hbm, val_hbm, zeros_hbm,
                    t_hbm, tbl_sh, ib, pb, vb, idxb, semS):
    c = lax.axis_index("c")
    s = lax.axis_index("s")
    nch = rows_per_tile // crows
    rowbase = s * rows_per_tile
    nbase = c * NPH

    def stage(i, p):
        ro = rowbase + i * crows
        pltpu.async_copy(ki_hbm.at[pl.ds(ro, crows)], ib.at[p], semS)
        pltpu.async_copy(kp_hbm.at[pl.ds(ro, crows)], pb.at[p], semS)
        pltpu.async_copy(val_hbm.at[pl.ds(ro, crows)], vb.at[p], semS)

    def drain_stage(i, p):
        ro = rowbase + i * crows
        pltpu.make_async_copy(ki_hbm.at[pl.ds(ro, crows)], ib.at[p], semS).wait()
        pltpu.make_async_copy(kp_hbm.at[pl.ds(ro, crows)], pb.at[p], semS).wait()
        pltpu.make_async_copy(val_hbm.at[pl.ds(ro, crows)], vb.at[p], semS).wait()

    stage(0, 0)
    pltpu.sync_copy(zeros_hbm, tbl_sh.at[pl.ds(s * _TLSZ, NP)])
    pltpu.sync_copy(zeros_hbm, tbl_sh.at[pl.ds(s * _TLSZ + NP, NP)])

    @pl.when(s == 0)
    def _zt():
        pltpu.sync_copy(zeros_hbm.at[pl.ds(0, 128)], tbl_sh.at[pl.ds(_TSZ, 128)])

    plsc.subcore_barrier()

    lanes = lax.broadcasted_iota(jnp.int32, (16,), 0)

    def body_one(i, p):
        drain_stage(i, p)

        @pl.when(i + 1 < nch)
        def _():
            stage(i + 1, 1 - p)

        for j in range(crows):
            for l in range(8):
                sl = pl.ds(l * 16, 16)
                i16 = ib[p, j, sl]
                p16 = pb[p, j, sl]
                loc = i16 - nbase
                valid = (loc >= 0) & (loc < NPH) & (p16 >= 0)
                idxb[j, sl] = jnp.where(valid, p16 * NPH + loc, _TSZ + lanes)
        for j in range(crows):
            pltpu.sync_copy(vb.at[p, j], tbl_sh.at[idxb.at[j]], add=True)

    def pair(k, carry):
        body_one(2 * k, 0)
        body_one(2 * k + 1, 1)
        return carry

    lax.fori_loop(0, nch // 2, pair, 0)
    plsc.subcore_barrier()
    pltpu.sync_copy(tbl_sh.at[pl.ds(s * _TLSZ, _TLSZ)],
                    t_hbm.at[pl.ds(c * _TSZ + s * _TLSZ, _TLSZ)])


def _sc_table2(ki2, kp2, val2, rows_per_tile, crows, zeros_np):
    f = pl.kernel(
        functools.partial(_sc_table2_body, rows_per_tile, crows),
        out_type=jax.ShapeDtypeStruct((2 * _TSZ,), jnp.float32),
        mesh=_MESH,
        compiler_params=pltpu.CompilerParams(needs_layout_passes=False),
        scratch_types=[
            pltpu.VMEM_SHARED((_TSZ + 128,), jnp.float32),
            pltpu.VMEM((2, crows, 128), jnp.int32),
            pltpu.VMEM((2, crows, 128), jnp.int32),
            pltpu.VMEM((2, crows, 128), jnp.float32),
            pltpu.VMEM((crows, 128), jnp.int32),
            pltpu.SemaphoreType.DMA,
        ],
    )
    t = f(ki2, kp2, val2, zeros_np).reshape(2, P, NPH)
    return jnp.concatenate([t[0], t[1]], axis=1)


_CRK = 16
_NCHK = EROWS // _CRK


def _sc_kbuild_body(md_hbm, ms_hbm, w_hbm, mc_hbm, zeros_hbm, k_hbm,
                    mc_sh, mdb, msb, wb, idx, g, acc0, semS, semG):
    c = lax.axis_index("c")
    s = lax.axis_index("s")

    def stage(i, p):
        ro = i * _CRK
        pltpu.async_copy(md_hbm.at[pl.ds(ro, _CRK)], mdb.at[p], semS)
        pltpu.async_copy(ms_hbm.at[pl.ds(ro, _CRK)], msb.at[p], semS)
        pltpu.async_copy(w_hbm.at[pl.ds(ro, _CRK)], wb.at[p], semS)

    def drain_stage(i, p):
        ro = i * _CRK
        pltpu.make_async_copy(md_hbm.at[pl.ds(ro, _CRK)], mdb.at[p], semS).wait()
        pltpu.make_async_copy(ms_hbm.at[pl.ds(ro, _CRK)], msb.at[p], semS).wait()
        pltpu.make_async_copy(w_hbm.at[pl.ds(ro, _CRK)], wb.at[p], semS).wait()

    for h in range(2):
        @pl.when(s == 0)
        def _load():
            pltpu.sync_copy(
                mc_hbm.at[pl.ds((c * 32 + h * 16) * NP, 16 * NP)], mc_sh)

        plsc.subcore_barrier()

        pltpu.sync_copy(zeros_hbm, acc0)
        off0 = s * NP
        p0 = c * 32 + h * 16 + s

        stage(0, 0)

        def body_one(i, p):
            drain_stage(i, p)
            for j in range(_CRK):
                for l in range(8):
                    sl = pl.ds(l * 16, 16)
                    idx[p, j, sl] = mdb[p, j, sl] + off0
            for j in range(_CRK):
                pltpu.async_copy(mc_sh.at[idx.at[p, j]], g.at[p, j], semG)

            @pl.when(i > 0)
            def _():
                q = 1 - p
                for j in range(_CRK):
                    pltpu.make_async_copy(mc_sh.at[idx.at[q, j]],
                                          g.at[q, j], semG).wait()
                for j in range(_CRK):
                    for l in range(8):
                        sl = pl.ds(l * 16, 16)
                        plsc.addupdate_scatter(
                            acc0, [msb[q, j, sl]], wb[q, j, sl] * g[q, j, sl])

            @pl.when(i + 1 < _NCHK)
            def _():
                stage(i + 1, 1 - p)

        def pair(k, carry):
            body_one(2 * k, 0)
            body_one(2 * k + 1, 1)
            return carry

        lax.fori_loop(0, _NCHK // 2, pair, 0)

        qe = 1
        for j in range(_CRK):
            pltpu.make_async_copy(mc_sh.at[idx.at[qe, j]],
                                  g.at[qe, j], semG).wait()
        for j in range(_CRK):
            for l in range(8):
                sl = pl.ds(l * 16, 16)
                plsc.addupdate_scatter(
                    acc0, [msb[qe, j, sl]], wb[qe, j, sl] * g[qe, j, sl])

        pltpu.sync_copy(acc0, k_hbm.at[pl.ds(p0 * NP, NP)])
        plsc.subcore_barrier()


def _sc_kbuild(md2, ms2, w2, mc_flat, zeros_np):
    f = pl.kernel(
        _sc_kbuild_body,
        out_type=jax.ShapeDtypeStruct((P * NP,), jnp.float32),
        mesh=_MESH,
        compiler_params=pltpu.CompilerParams(needs_layout_passes=False),
        scratch_types=[
            pltpu.VMEM_SHARED((16 * NP,), jnp.float32),
            pltpu.VMEM((2, _CRK, 128), jnp.int32),
            pltpu.VMEM((2, _CRK, 128), jnp.int32),
            pltpu.VMEM((2, _CRK, 128), jnp.float32),
            pltpu.VMEM((2, _CRK, 128), jnp.int32),
            pltpu.VMEM((2, _CRK, 128), jnp.float32),
            pltpu.VMEM((NP,), jnp.float32),
            pltpu.SemaphoreType.DMA,
            pltpu.SemaphoreType.DMA,
        ],
    )
    return f(md2, ms2, w2, mc_flat, zeros_np).reshape(P, NP)


def _tc_reduce_body(g_ref, bmat_ref, ct_ref, x_ref, s2x_ref,
                    w0_ref, b0_ref, wu_ref, bu_ref,
                    mc_ref, sg_ref, p1_ref, mcct_ref, smalls_ref, r_ref):
    i = pl.program_id(0)

    gblk = g_ref[...]
    bblk = bmat_ref[...]
    ctblk = ct_ref[...]
    xblk = x_ref[...]
    s2xblk = s2x_ref[...]

    mblk = gblk + bblk
    cm = jnp.maximum(jnp.sum(ctblk, axis=0, keepdims=True), 1.0)
    mcblk = mblk / cm
    mc_ref[...] = mcblk

    sg = jnp.dot(mblk, xblk, preferred_element_type=jnp.float32)
    p1 = jnp.dot(mcblk, s2xblk, preferred_element_type=jnp.float32)
    mcct = lax.dot_general(mcblk, ctblk, (((1,), (1,)), ((), ())),
                           preferred_element_type=jnp.float32)

    lane = lax.broadcasted_iota(jnp.int32, (P, D), 1)
    rs_m = jnp.sum(mblk, axis=1)
    rs_ct = jnp.sum(ctblk, axis=1)
    smalls = (jnp.where(lane == 0, rs_m[:, None], 0.0)
              + jnp.where(lane == 1, rs_ct[:, None], 0.0))

    @pl.when(i == 0)
    def _init():
        sg_ref[...] = sg
        p1_ref[...] = p1
        mcct_ref[...] = mcct
        smalls_ref[...] = smalls

    @pl.when(i > 0)
    def _acc():
        sg_ref[...] += sg
        p1_ref[...] += p1
        mcct_ref[...] += mcct
        smalls_ref[...] += smalls

    @pl.when(i == GRID - 1)
    def _epilogue():
        sm = smalls_ref[...]
        cnt_b = jnp.sum(jnp.where(lane == 1, sm, 0.0), axis=1)
        cb = jnp.maximum(cnt_b, 1.0)
        sgpool = (jnp.dot(sg_ref[...] / cb[:, None], w0_ref[...],
                          preferred_element_type=jnp.float32)
                  + b0_ref[...])
        r_ref[...] = jax.nn.relu(
            jnp.dot(sgpool, wu_ref[...], preferred_element_type=jnp.float32)
            + bu_ref[...])


def _tc_reduce(G, B, Ct, x, s2x, W0, b0, Wu, bu):
    tbl = pl.BlockSpec((P, BN), lambda i: (0, i))
    xsp = pl.BlockSpec((BN, D), lambda i: (i, 0))
    wsp = pl.BlockSpec((D, D), lambda i: (0, 0))
    bsp = pl.BlockSpec((1, D), lambda i: (0, 0))
    acc = pl.BlockSpec((P, D), lambda i: (0, 0))
    out_shapes = (
        jax.ShapeDtypeStruct((P, NP), jnp.float32),
        jax.ShapeDtypeStruct((P, D), jnp.float32),
        jax.ShapeDtypeStruct((P, D), jnp.float32),
        jax.ShapeDtypeStruct((P, P), jnp.float32),
        jax.ShapeDtypeStruct((P, D), jnp.float32),
        jax.ShapeDtypeStruct((P, D), jnp.float32),
    )
    return pl.pallas_call(
        _tc_reduce_body,
        grid=(GRID,),
        in_specs=[tbl, tbl, tbl, xsp, xsp, wsp, bsp, wsp, bsp],
        out_specs=(tbl, acc, acc, pl.BlockSpec((P, P), lambda i: (0, 0)),
                   acc, acc),
        out_shape=out_shapes,
        compiler_params=pltpu.CompilerParams(
            dimension_semantics=("arbitrary",)),
    )(G, B, Ct, x, s2x, W0, b0, Wu, bu)


def _tc_out_body(k_ref, x_ref, p1_ref, mcct_ref, r_ref, smalls_ref,
                 w0_ref, b0_ref, w1_ref, b1_ref,
                 kx_ref, out_ref):
    i = pl.program_id(0)
    kx = jnp.dot(k_ref[...], x_ref[...], preferred_element_type=jnp.float32)

    @pl.when(i == 0)
    def _init():
        kx_ref[...] = kx

    @pl.when(i > 0)
    def _acc():
        kx_ref[...] += kx

    @pl.when(i == GRID - 1)
    def _epilogue():
        lane = lax.broadcasted_iota(jnp.int32, (P, D), 1)
        sm = smalls_ref[...]
        colsum_m = jnp.sum(jnp.where(lane == 0, sm, 0.0), axis=1)
        cnt_b = jnp.sum(jnp.where(lane == 1, sm, 0.0), axis=1)
        cb = jnp.maximum(cnt_b, 1.0)
        out_sum = (jnp.dot(kx_ref[...] + p1_ref[...], w0_ref[...],
                           preferred_element_type=jnp.float32)
                   + jnp.dot(mcct_ref[...], r_ref[...],
                             preferred_element_type=jnp.float32)
                   + colsum_m[:, None] * b0_ref[...])
        out = (jnp.dot(out_sum / cb[:, None], w1_ref[...],
                       preferred_element_type=jnp.float32)
               + b1_ref[...])
        out_ref[...] = jnp.where(cnt_b[:, None] > 0, out, 0.0)


def _tc_out(K, x, partial1, McCt, r, smalls, W0, b0, W1, b1):
    tbl = pl.BlockSpec((P, BN), lambda i: (0, i))
    xsp = pl.BlockSpec((BN, D), lambda i: (i, 0))
    wsp = pl.BlockSpec((D, D), lambda i: (0, 0))
    bsp = pl.BlockSpec((1, D), lambda i: (0, 0))
    acc = pl.BlockSpec((P, D), lambda i: (0, 0))
    out_shapes = (
        jax.ShapeDtypeStruct((P, D), jnp.float32),
        jax.ShapeDtypeStruct((P, D), jnp.float32),
    )
    res = pl.pallas_call(
        _tc_out_body,
        grid=(GRID,),
        in_specs=[tbl, xsp, acc, pl.BlockSpec((P, P), lambda i: (0, 0)),
                  acc, acc, wsp, bsp, wsp, bsp],
        out_specs=(acc, acc),
        out_shape=out_shapes,
        compiler_params=pltpu.CompilerParams(
            dimension_semantics=("arbitrary",)),
    )(K, x, partial1, McCt, r, smalls, W0, b0, W1, b1)
    return res[1]


def kernel(x, edge_attr, subgraphs_nodes_mapper, subgraphs_edges_mapper,
           combined_subgraphs, subgraphs_batch, W0, b0, W1, b1, Wu, bu):
    m = subgraphs_nodes_mapper
    b = subgraphs_batch
    src = combined_subgraphs[0]
    dst = combined_subgraphs[1]

    zeros_np = jnp.zeros((NP,), jnp.float32)

    src2 = jnp.pad(src, (0, E_PAD - E_SUB),
                   constant_values=N_SUB).reshape(EROWS, 128)
    dst2 = jnp.pad(dst, (0, E_PAD - E_SUB),
                   constant_values=N_SUB).reshape(EROWS, 128)
    m_tbl = jnp.pad(m, (0, S_TBL - N_SUB), constant_values=NP - 1)
    b_tbl = jnp.pad(b, (0, S_TBL - N_SUB), constant_values=-1)

    ms2, md2, bd2, indeg2 = _sc_gather_maps(src2, dst2, m_tbl, b_tbl, zeros_np)
    indeg = indeg2.reshape(2, S_TBL).sum(axis=0)

    deg = 1.0 + indeg[:N_SUB]
    dinv = lax.rsqrt(deg)
    dinv2 = 1.0 / deg
    dinv_tbl = jnp.pad(dinv, (0, S_TBL - N_SUB))

    m_t2 = jnp.pad(m, (0, S_STREAM2 - N_SUB),
                   constant_values=-1).reshape(SROWS2, 128)
    dv2_t2 = jnp.pad(dinv2, (0, S_STREAM2 - N_SUB)).reshape(SROWS2, 128)

    w2, s2p = _sc_edge_weights(src2, dst2, dinv_tbl, m_t2, dv2_t2, zeros_np)
    s2 = s2p.reshape(2, NP).sum(axis=0)

    b_t2 = jnp.pad(b, (0, S_STREAM2 - N_SUB),
                   constant_values=-1).reshape(SROWS2, 128)
    ones_t2 = jnp.ones((SROWS2, 128), jnp.float32)

    G = _sc_table2(ms2, bd2, w2, EROWS // 16, 24, zeros_np)
    Bm = _sc_table2(m_t2, b_t2, dv2_t2, SROWS2 // 16, 8, zeros_np)
    Ct = _sc_table2(m_t2, b_t2, ones_t2, SROWS2 // 16, 8, zeros_np)

    xp = jnp.pad(x, ((0, NP - N_NODES), (0, 0)))
    s2x = xp * s2[:, None]

    Mc, sg_sum, partial1, McCt, smalls, r = _tc_reduce(
        G, Bm, Ct, xp, s2x, W0, b0[None, :], Wu, bu[None, :])

    K = _sc_kbuild(md2, ms2, w2, Mc.reshape(P * NP), zeros_np)

    return _tc_out(K, xp, partial1, McCt, r, smalls, W0, b0[None, :],
                   W1, b1[None, :])


# speedup vs baseline: 40.1200x; 1.0004x over previous
import functools
import jax
import jax.numpy as jnp
from jax import lax
from jax.experimental import pallas as pl
from jax.experimental.pallas import tpu as pltpu
from jax.experimental.pallas import tpu_sc as plsc

N_NODES = 50000
N_SUB = 150000
E_SUB = 480000
D = 128
P = 64

NP = 51200
BN = 1024
GRID = NP // BN

E_PAD = 491520
EROWS = E_PAD // 128
EW_ROWS = EROWS // 32
S_TBL = 150016
S_STREAM = 150528
SROWS = S_STREAM // 128
S_STREAM2 = 163840
SROWS2 = S_STREAM2 // 128
KW_I = S_TBL // 32
KW_S = NP // 32
_CR = 24

_MESH = plsc.VectorSubcoreMesh(core_axis_name="c", subcore_axis_name="s",
                               num_cores=2, num_subcores=16)


def _wid():
    return lax.axis_index("c") * 16 + lax.axis_index("s")


def _sc_gather_maps_body(src_hbm, dst_hbm, mtbl_hbm, btbl_hbm, zeros2_hbm,
                         ms_hbm, md_hbm, bd_hbm, indeg_hbm,
                         m_sh, b_sh, ideg_sh, sbuf, dbuf, msv, mdv, bdv,
                         onesb, sem):
    s = lax.axis_index("s")
    wid = _wid()

    @pl.when(s == 0)
    def _load():
        pltpu.sync_copy(mtbl_hbm, m_sh)
        pltpu.sync_copy(btbl_hbm, b_sh)

    plsc.subcore_barrier()

    rowbase = wid * EW_ROWS

    def chunk1(ci, carry):
        ro = rowbase + ci * _CR
        pltpu.sync_copy(src_hbm.at[pl.ds(ro, _CR)], sbuf)
        pltpu.sync_copy(dst_hbm.at[pl.ds(ro, _CR)], dbuf)
        hs = []
        for j in range(_CR):
            hs.append(pltpu.async_copy(m_sh.at[sbuf.at[j]], msv.at[j], sem))
            hs.append(pltpu.async_copy(m_sh.at[dbuf.at[j]], mdv.at[j], sem))
            hs.append(pltpu.async_copy(b_sh.at[dbuf.at[j]], bdv.at[j], sem))
        for h in hs:
            h.wait()
        pltpu.sync_copy(msv, ms_hbm.at[pl.ds(ro, _CR)])
        pltpu.sync_copy(mdv, md_hbm.at[pl.ds(ro, _CR)])
        pltpu.sync_copy(bdv, bd_hbm.at[pl.ds(ro, _CR)])
        return carry

    lax.fori_loop(0, EW_ROWS // _CR, chunk1, 0)

    c = lax.axis_index("c")
    for l in range(8):
        onesb[0, pl.ds(l * 16, 16)] = jnp.ones((16,), jnp.float32)
    zch = S_TBL // 4
    @pl.when(s < 4)
    def _zi():
        pltpu.sync_copy(zeros2_hbm.at[pl.ds(0, zch)],
                        ideg_sh.at[pl.ds(s * zch, zch)])
    plsc.subcore_barrier()

    rb2 = c * (EROWS // 2) + s * (EROWS // 32)

    def chunk2(ci, carry):
        pltpu.sync_copy(dst_hbm.at[pl.ds(rb2 + ci * _CR, _CR)], dbuf)
        for j in range(_CR):
            pltpu.sync_copy(onesb.at[0], ideg_sh.at[dbuf.at[j]], add=True)
        return carry

    lax.fori_loop(0, EROWS // 32 // _CR, chunk2, 0)
    plsc.subcore_barrier()

    @pl.when(s < 4)
    def _wb():
        pltpu.sync_copy(ideg_sh.at[pl.ds(s * zch, zch)],
                        indeg_hbm.at[pl.ds(c * S_TBL + s * zch, zch)])


def _sc_gather_maps(src2, dst2, m_tbl, b_tbl, zeros_np):
    f = pl.kernel(
        _sc_gather_maps_body,
        out_type=(
            jax.ShapeDtypeStruct((EROWS, 128), jnp.int32),
            jax.ShapeDtypeStruct((EROWS, 128), jnp.int32),
            jax.ShapeDtypeStruct((EROWS, 128), jnp.int32),
            jax.ShapeDtypeStruct((2 * S_TBL,), jnp.float32),
        ),
        mesh=_MESH,
        compiler_params=pltpu.CompilerParams(needs_layout_passes=False),
        scratch_types=[
            pltpu.VMEM_SHARED((S_TBL,), jnp.int32),
            pltpu.VMEM_SHARED((S_TBL,), jnp.int32),
            pltpu.VMEM_SHARED((S_TBL,), jnp.float32),
            pltpu.VMEM((_CR, 128), jnp.int32),
            pltpu.VMEM((_CR, 128), jnp.int32),
            pltpu.VMEM((_CR, 128), jnp.int32),
            pltpu.VMEM((_CR, 128), jnp.int32),
            pltpu.VMEM((_CR, 128), jnp.int32),
            pltpu.VMEM((1, 128), jnp.float32),
            pltpu.SemaphoreType.DMA,
        ],
    )
    return f(src2, dst2, m_tbl, b_tbl, zeros_np)


def _sc_edge_weights_body(src_hbm, dst_hbm, dinv_hbm, ms2_hbm, dv2s_hbm,
                          zeros_hbm,
                          w_hbm, s2_hbm,
                          dinv_sh, s2_sh, sbuf, dbuf, va, vb, wbuf, mbuf,
                          vbuf, idxb2, sem):
    s = lax.axis_index("s")
    wid = _wid()

    @pl.when(s == 0)
    def _load():
        pltpu.sync_copy(dinv_hbm, dinv_sh)

    plsc.subcore_barrier()

    rowbase = wid * EW_ROWS

    def chunk1(ci, carry):
        ro = rowbase + ci * _CR
        pltpu.sync_copy(src_hbm.at[pl.ds(ro, _CR)], sbuf)
        pltpu.sync_copy(dst_hbm.at[pl.ds(ro, _CR)], dbuf)
        hs = []
        for j in range(_CR):
            hs.append(pltpu.async_copy(dinv_sh.at[sbuf.at[j]], va.at[j], sem))
            hs.append(pltpu.async_copy(dinv_sh.at[dbuf.at[j]], vb.at[j], sem))
        for h in hs:
            h.wait()
        for j in range(_CR):
            for l in range(8):
                sl = pl.ds(l * 16, 16)
                wbuf[j, sl] = va[j, sl] * vb[j, sl]
        pltpu.sync_copy(wbuf, w_hbm.at[pl.ds(ro, _CR)])
        return carry

    lax.fori_loop(0, EW_ROWS // _CR, chunk1, 0)

    c = lax.axis_index("c")
    lanes = lax.broadcasted_iota(jnp.int32, (16,), 0)
    psz = NP // 16
    pltpu.sync_copy(zeros_hbm.at[pl.ds(0, psz)],
                    s2_sh.at[pl.ds(s * psz, psz)])

    @pl.when(s == 0)
    def _zt():
        pltpu.sync_copy(zeros_hbm.at[pl.ds(0, 128)], s2_sh.at[pl.ds(NP, 128)])

    plsc.subcore_barrier()

    rb2 = c * (SROWS2 // 2) + s * (SROWS2 // 32)

    def chunk2(ci, carry):
        ro = rb2 + ci * 8
        pltpu.sync_copy(ms2_hbm.at[pl.ds(ro, 8)], mbuf)
        pltpu.sync_copy(dv2s_hbm.at[pl.ds(ro, 8)], vbuf)
        for j in range(8):
            for l in range(8):
                sl = pl.ds(l * 16, 16)
                i16 = mbuf[j, sl]
                idxb2[j, sl] = jnp.where(i16 >= 0, i16, NP + lanes)
        for j in range(8):
            pltpu.sync_copy(vbuf.at[j], s2_sh.at[idxb2.at[j]], add=True)
        return carry

    lax.fori_loop(0, SROWS2 // 32 // 8, chunk2, 0)
    plsc.subcore_barrier()
    pltpu.sync_copy(s2_sh.at[pl.ds(s * psz, psz)],
                    s2_hbm.at[pl.ds(c * NP + s * psz, psz)])


def _sc_edge_weights(src2, dst2, dinv_tbl, m_s2, dv2_s2, zeros_np):
    f = pl.kernel(
        _sc_edge_weights_body,
        out_type=(
            jax.ShapeDtypeStruct((EROWS, 128), jnp.float32),
            jax.ShapeDtypeStruct((2 * NP,), jnp.float32),
        ),
        mesh=_MESH,
        compiler_params=pltpu.CompilerParams(needs_layout_passes=False),
        scratch_types=[
            pltpu.VMEM_SHARED((S_TBL,), jnp.float32),
            pltpu.VMEM_SHARED((NP + 128,), jnp.float32),
            pltpu.VMEM((_CR, 128), jnp.int32),
            pltpu.VMEM((_CR, 128), jnp.int32),
            pltpu.VMEM((_CR, 128), jnp.float32),
            pltpu.VMEM((_CR, 128), jnp.float32),
            pltpu.VMEM((_CR, 128), jnp.float32),
            pltpu.VMEM((8, 128), jnp.int32),
            pltpu.VMEM((8, 128), jnp.float32),
            pltpu.VMEM((8, 128), jnp.int32),
            pltpu.SemaphoreType.DMA,
        ],
    )
    return f(src2, dst2, dinv_tbl, m_s2, dv2_s2, zeros_np)


NPH = NP // 2
_TSZ = P * NPH
_TLSZ = _TSZ // 16


def _sc_table2_body(rows_per_tile, crows, ki_hbm, kp_hbm, val_hbm, zeros_hbm,
                    t_hbm, tbl_sh, ib, pb, vb, idxb, semS):
    c = lax.axis_index("c")
    s = lax.axis_index("s")
    nch = rows_per_tile // crows
    rowbase = s * rows_per_tile
    nbase = c * NPH

    def stage(i, p):
        ro = rowbase + i * crows
        pltpu.async_copy(ki_hbm.at[pl.ds(ro, crows)], ib.at[p], semS)
        pltpu.async_copy(kp_hbm.at[pl.ds(ro, crows)], pb.at[p], semS)
        pltpu.async_copy(val_hbm.at[pl.ds(ro, crows)], vb.at[p], semS)

    def drain_stage(i, p):
        ro = rowbase + i * crows
        pltpu.make_async_copy(ki_hbm.at[pl.ds(ro, crows)], ib.at[p], semS).wait()
        pltpu.make_async_copy(kp_hbm.at[pl.ds(ro, crows)], pb.at[p], semS).wait()
        pltpu.make_async_copy(val_hbm.at[pl.ds(ro, crows)], vb.at[p], semS).wait()

    stage(0, 0)
    pltpu.sync_copy(zeros_hbm, tbl_sh.at[pl.ds(s * _TLSZ, NP)])
    pltpu.sync_copy(zeros_hbm, tbl_sh.at[pl.ds(s * _TLSZ + NP, NP)])

    @pl.when(s == 0)
    def _zt():
        pltpu.sync_copy(zeros_hbm.at[pl.ds(0, 128)], tbl_sh.at[pl.ds(_TSZ, 128)])

    plsc.subcore_barrier()

    lanes = lax.broadcasted_iota(jnp.int32, (16,), 0)

    def body_one(i, p):
        drain_stage(i, p)

        @pl.when(i + 1 < nch)
        def _():
            stage(i + 1, 1 - p)

        for j in range(crows):
            for l in range(8):
                sl = pl.ds(l * 16, 16)
                i16 = ib[p, j, sl]
                p16 = pb[p, j, sl]
                loc = i16 - nbase
                valid = (loc >= 0) & (loc < NPH) & (p16 >= 0)
                idxb[j, sl] = jnp.where(valid, p16 * NPH + loc, _TSZ + lanes)
        for j in range(crows):
            pltpu.sync_copy(vb.at[p, j], tbl_sh.at[idxb.at[j]], add=True)

    def pair(k, carry):
        body_one(2 * k, 0)
        body_one(2 * k + 1, 1)
        return carry

    lax.fori_loop(0, nch // 2, pair, 0)
    plsc.subcore_barrier()
    pltpu.sync_copy(tbl_sh.at[pl.ds(s * _TLSZ, _TLSZ)],
                    t_hbm.at[pl.ds(c * _TSZ + s * _TLSZ, _TLSZ)])


def _sc_table2(ki2, kp2, val2, rows_per_tile, crows, zeros_np):
    f = pl.kernel(
        functools.partial(_sc_table2_body, rows_per_tile, crows),
        out_type=jax.ShapeDtypeStruct((2 * _TSZ,), jnp.float32),
        mesh=_MESH,
        compiler_params=pltpu.CompilerParams(needs_layout_passes=False),
        scratch_types=[
            pltpu.VMEM_SHARED((_TSZ + 128,), jnp.float32),
            pltpu.VMEM((2, crows, 128), jnp.int32),
            pltpu.VMEM((2, crows, 128), jnp.int32),
            pltpu.VMEM((2, crows, 128), jnp.float32),
            pltpu.VMEM((crows, 128), jnp.int32),
            pltpu.SemaphoreType.DMA,
        ],
    )
    t = f(ki2, kp2, val2, zeros_np).reshape(2, P, NPH)
    return jnp.concatenate([t[0], t[1]], axis=1)


_CRK = 16
_NCHK = EROWS // _CRK


def _sc_kbuild_body(md_hbm, ms_hbm, w_hbm, mc_hbm, zeros_hbm, k_hbm,
                    mc_sh, mdb, msb, wb, idx, g, acc0, semS, semG):
    c = lax.axis_index("c")
    s = lax.axis_index("s")

    def stage(i, p):
        ro = i * _CRK
        pltpu.async_copy(md_hbm.at[pl.ds(ro, _CRK)], mdb.at[p], semS)
        pltpu.async_copy(ms_hbm.at[pl.ds(ro, _CRK)], msb.at[p], semS)
        pltpu.async_copy(w_hbm.at[pl.ds(ro, _CRK)], wb.at[p], semS)

    def drain_stage(i, p):
        ro = i * _CRK
        pltpu.make_async_copy(md_hbm.at[pl.ds(ro, _CRK)], mdb.at[p], semS).wait()
        pltpu.make_async_copy(ms_hbm.at[pl.ds(ro, _CRK)], msb.at[p], semS).wait()
        pltpu.make_async_copy(w_hbm.at[pl.ds(ro, _CRK)], wb.at[p], semS).wait()

    for h in range(2):
        @pl.when(s == 0)
        def _load():
            pltpu.sync_copy(
                mc_hbm.at[pl.ds((c * 32 + h * 16) * NP, 16 * NP)], mc_sh)

        plsc.subcore_barrier()

        pltpu.sync_copy(zeros_hbm, acc0)
        off0 = s * NP
        p0 = c * 32 + h * 16 + s

        stage(0, 0)

        def body_one(i, p):
            drain_stage(i, p)
            for j in range(_CRK):
                for l in range(8):
                    sl = pl.ds(l * 16, 16)
                    idx[p, j, sl] = mdb[p, j, sl] + off0
            for j in range(_CRK):
                pltpu.async_copy(mc_sh.at[idx.at[p, j]], g.at[p, j], semG)

            @pl.when(i > 0)
            def _():
                q = 1 - p
                for j in range(_CRK):
                    pltpu.make_async_copy(mc_sh.at[idx.at[q, j]],
                                          g.at[q, j], semG).wait()
                for j in range(_CRK):
                    for l in range(8):
                        sl = pl.ds(l * 16, 16)
                        plsc.addupdate_scatter(
                            acc0, [msb[q, j, sl]], wb[q, j, sl] * g[q, j, sl])

            @pl.when(i + 1 < _NCHK)
            def _():
                stage(i + 1, 1 - p)

        def pair(k, carry):
            body_one(2 * k, 0)
            body_one(2 * k + 1, 1)
            return carry

        lax.fori_loop(0, _NCHK // 2, pair, 0)

        qe = 1
        for j in range(_CRK):
            pltpu.make_async_copy(mc_sh.at[idx.at[qe, j]],
                                  g.at[qe, j], semG).wait()
        for j in range(_CRK):
            for l in range(8):
                sl = pl.ds(l * 16, 16)
                plsc.addupdate_scatter(
                    acc0, [msb[qe, j, sl]], wb[qe, j, sl] * g[qe, j, sl])

        pltpu.sync_copy(acc0, k_hbm.at[pl.ds(p0 * NP, NP)])
        plsc.subcore_barrier()


def _sc_kbuild(md2, ms2, w2, mc_flat, zeros_np):
    f = pl.kernel(
        _sc_kbuild_body,
        out_type=jax.ShapeDtypeStruct((P * NP,), jnp.float32),
        mesh=_MESH,
        compiler_params=pltpu.CompilerParams(needs_layout_passes=False),
        scratch_types=[
            pltpu.VMEM_SHARED((16 * NP,), jnp.float32),
            pltpu.VMEM((2, _CRK, 128), jnp.int32),
            pltpu.VMEM((2, _CRK, 128), jnp.int32),
            pltpu.VMEM((2, _CRK, 128), jnp.float32),
            pltpu.VMEM((2, _CRK, 128), jnp.int32),
            pltpu.VMEM((2, _CRK, 128), jnp.float32),
            pltpu.VMEM((NP,), jnp.float32),
            pltpu.SemaphoreType.DMA,
            pltpu.SemaphoreType.DMA,
        ],
    )
    return f(md2, ms2, w2, mc_flat, zeros_np).reshape(P, NP)


def _tc_reduce_body(g_ref, bmat_ref, ct_ref, x_ref, s2x_ref,
                    w0_ref, b0_ref, wu_ref, bu_ref,
                    mc_ref, sg_ref, p1_ref, mcct_ref, smalls_ref, r_ref):
    i = pl.program_id(0)

    gblk = g_ref[...]
    bblk = bmat_ref[...]
    ctblk = ct_ref[...]
    xblk = x_ref[...]
    s2xblk = s2x_ref[...]

    mblk = gblk + bblk
    cm = jnp.maximum(jnp.sum(ctblk, axis=0, keepdims=True), 1.0)
    mcblk = mblk / cm
    mc_ref[...] = mcblk

    sg = jnp.dot(mblk, xblk, preferred_element_type=jnp.float32)
    p1 = jnp.dot(mcblk, s2xblk, preferred_element_type=jnp.float32)
    mcct = lax.dot_general(mcblk, ctblk, (((1,), (1,)), ((), ())),
                           preferred_element_type=jnp.float32)

    lane = lax.broadcasted_iota(jnp.int32, (P, D), 1)
    rs_m = jnp.sum(mblk, axis=1)
    rs_ct = jnp.sum(ctblk, axis=1)
    smalls = (jnp.where(lane == 0, rs_m[:, None], 0.0)
              + jnp.where(lane == 1, rs_ct[:, None], 0.0))

    @pl.when(i == 0)
    def _init():
        sg_ref[...] = sg
        p1_ref[...] = p1
        mcct_ref[...] = mcct
        smalls_ref[...] = smalls

    @pl.when(i > 0)
    def _acc():
        sg_ref[...] += sg
        p1_ref[...] += p1
        mcct_ref[...] += mcct
        smalls_ref[...] += smalls

    @pl.when(i == GRID - 1)
    def _epilogue():
        sm = smalls_ref[...]
        cnt_b = jnp.sum(jnp.where(lane == 1, sm, 0.0), axis=1)
        cb = jnp.maximum(cnt_b, 1.0)
        sgpool = (jnp.dot(sg_ref[...] / cb[:, None], w0_ref[...],
                          preferred_element_type=jnp.float32)
                  + b0_ref[...])
        r_ref[...] = jax.nn.relu(
            jnp.dot(sgpool, wu_ref[...], preferred_element_type=jnp.float32)
            + bu_ref[...])


def _tc_reduce(G, B, Ct, x, s2x, W0, b0, Wu, bu):
    tbl = pl.BlockSpec((P, BN), lambda i: (0, i))
    xsp = pl.BlockSpec((BN, D), lambda i: (i, 0))
    wsp = pl.BlockSpec((D, D), lambda i: (0, 0))
    bsp = pl.BlockSpec((1, D), lambda i: (0, 0))
    acc = pl.BlockSpec((P, D), lambda i: (0, 0))
    out_shapes = (
        jax.ShapeDtypeStruct((P, NP), jnp.float32),
        jax.ShapeDtypeStruct((P, D), jnp.float32),
        jax.ShapeDtypeStruct((P, D), jnp.float32),
        jax.ShapeDtypeStruct((P, P), jnp.float32),
        jax.ShapeDtypeStruct((P, D), jnp.float32),
        jax.ShapeDtypeStruct((P, D), jnp.float32),
    )
    return pl.pallas_call(
        _tc_reduce_body,
        grid=(GRID,),
        in_specs=[tbl, tbl, tbl, xsp, xsp, wsp, bsp, wsp, bsp],
        out_specs=(tbl, acc, acc, pl.BlockSpec((P, P), lambda i: (0, 0)),
                   acc, acc),
        out_shape=out_shapes,
        compiler_params=pltpu.CompilerParams(
            dimension_semantics=("arbitrary",)),
    )(G, B, Ct, x, s2x, W0, b0, Wu, bu)


def _tc_out_body(k_ref, x_ref, p1_ref, mcct_ref, r_ref, smalls_ref,
                 w0_ref, b0_ref, w1_ref, b1_ref,
                 kx_ref, out_ref):
    i = pl.program_id(0)
    kx = jnp.dot(k_ref[...], x_ref[...], preferred_element_type=jnp.float32)

    @pl.when(i == 0)
    def _init():
        kx_ref[...] = kx

    @pl.when(i > 0)
    def _acc():
        kx_ref[...] += kx

    @pl.when(i == GRID - 1)
    def _epilogue():
        lane = lax.broadcasted_iota(jnp.int32, (P, D), 1)
        sm = smalls_ref[...]
        colsum_m = jnp.sum(jnp.where(lane == 0, sm, 0.0), axis=1)
        cnt_b = jnp.sum(jnp.where(lane == 1, sm, 0.0), axis=1)
        cb = jnp.maximum(cnt_b, 1.0)
        out_sum = (jnp.dot(kx_ref[...] + p1_ref[...], w0_ref[...],
                           preferred_element_type=jnp.float32)
                   + jnp.dot(mcct_ref[...], r_ref[...],
                             preferred_element_type=jnp.float32)
                   + colsum_m[:, None] * b0_ref[...])
        out = (jnp.dot(out_sum / cb[:, None], w1_ref[...],
                       preferred_element_type=jnp.float32)
               + b1_ref[...])
        out_ref[...] = jnp.where(cnt_b[:, None] > 0, out, 0.0)


def _tc_out(K, x, partial1, McCt, r, smalls, W0, b0, W1, b1):
    tbl = pl.BlockSpec((P, BN), lambda i: (0, i))
    xsp = pl.BlockSpec((BN, D), lambda i: (i, 0))
    wsp = pl.BlockSpec((D, D), lambda i: (0, 0))
    bsp = pl.BlockSpec((1, D), lambda i: (0, 0))
    acc = pl.BlockSpec((P, D), lambda i: (0, 0))
    out_shapes = (
        jax.ShapeDtypeStruct((P, D), jnp.float32),
        jax.ShapeDtypeStruct((P, D), jnp.float32),
    )
    res = pl.pallas_call(
        _tc_out_body,
        grid=(GRID,),
        in_specs=[tbl, xsp, acc, pl.BlockSpec((P, P), lambda i: (0, 0)),
                  acc, acc, wsp, bsp, wsp, bsp],
        out_specs=(acc, acc),
        out_shape=out_shapes,
        compiler_params=pltpu.CompilerParams(
            dimension_semantics=("arbitrary",)),
    )(K, x, partial1, McCt, r, smalls, W0, b0, W1, b1)
    return res[1]


def kernel(x, edge_attr, subgraphs_nodes_mapper, subgraphs_edges_mapper,
           combined_subgraphs, subgraphs_batch, W0, b0, W1, b1, Wu, bu):
    m = subgraphs_nodes_mapper
    b = subgraphs_batch
    src = combined_subgraphs[0]
    dst = combined_subgraphs[1]

    zeros_np = jnp.zeros((NP,), jnp.float32)

    src2 = jnp.pad(src, (0, E_PAD - E_SUB),
                   constant_values=N_SUB).reshape(EROWS, 128)
    dst2 = jnp.pad(dst, (0, E_PAD - E_SUB),
                   constant_values=N_SUB).reshape(EROWS, 128)
    m_tbl = jnp.pad(m, (0, S_TBL - N_SUB), constant_values=NP - 1)
    b_tbl = jnp.pad(b, (0, S_TBL - N_SUB), constant_values=-1)

    ms2, md2, bd2, indeg2 = _sc_gather_maps(src2, dst2, m_tbl, b_tbl, zeros_np)
    indeg = indeg2.reshape(2, S_TBL).sum(axis=0)

    deg = 1.0 + indeg[:N_SUB]
    dinv = lax.rsqrt(deg)
    dinv2 = 1.0 / deg
    dinv_tbl = jnp.pad(dinv, (0, S_TBL - N_SUB))

    m_t2 = jnp.pad(m, (0, S_STREAM2 - N_SUB),
                   constant_values=-1).reshape(SROWS2, 128)
    dv2_t2 = jnp.pad(dinv2, (0, S_STREAM2 - N_SUB)).reshape(SROWS2, 128)

    w2, s2p = _sc_edge_weights(src2, dst2, dinv_tbl, m_t2, dv2_t2, zeros_np)
    s2 = s2p.reshape(2, NP).sum(axis=0)

    b_t2 = jnp.pad(b, (0, S_STREAM2 - N_SUB),
                   constant_values=-1).reshape(SROWS2, 128)
    ones_t2 = jnp.ones((SROWS2, 128), jnp.float32)

    G = _sc_table2(ms2, bd2, w2, EROWS // 16, 24, zeros_np)
    Bm = _sc_table2(m_t2, b_t2, dv2_t2, SROWS2 // 16, 8, zeros_np)
    Ct = _sc_table2(m_t2, b_t2, ones_t2, SROWS2 // 16, 8, zeros_np)

    xp = jnp.pad(x, ((0, NP - N_NODES), (0, 0)))
    s2x = xp * s2[:, None]

    Mc, sg_sum, partial1, McCt, smalls, r = _tc_reduce(
        G, Bm, Ct, xp, s2x, W0, b0[None, :], Wu, bu[None, :])

    K = _sc_kbuild(md2, ms2, w2, Mc.reshape(P * NP), zeros_np)

    return _tc_out(K, xp, partial1, McCt, r, smalls, W0, b0[None, :],
                   W1, b1[None, :])
